# Initial kernel scaffold; baseline (speedup 1.0000x reference)
import jax
import jax.numpy as jnp
from jax.experimental import pallas as pl


def kernel(x, W1, b1, W2, b2, l1W, l1b, l2W, l2b, c3W, c3b, c3g, c3be, m1W1, m1b1, m1g1, m1be1, m1W2, m1b2, m1g2, m1be2):
    raise NotImplementedError("write your pallas kernel here")


# speedup vs baseline: 3.5200x; 3.5200x over previous
import functools

import jax
import jax.numpy as jnp
from jax.experimental import pallas as pl
from jax.experimental.pallas import tpu as pltpu

KNN = 33
NKEY = 10
BIG = 3e38
F32 = jnp.float32
BF = jnp.bfloat16


def _split3(x):
    hi = x.astype(BF)
    r1 = x - hi.astype(F32)
    mid = r1.astype(BF)
    lo = (r1 - mid.astype(F32)).astype(BF)
    return hi, mid, lo


def _leaky(x):
    return jnp.where(x >= 0, x, 0.2 * x)


def _dot(a, b):
    return jnp.dot(a, b, preferred_element_type=F32)


def _knn_edge_body(R, N, F, emit_knn, xr_ref, xt_ref, xc_ref, W_ref, b_ref,
                   *out_refs):
    if emit_knn:
        dist_ref, idx_ref, xo_ref = out_refs
    else:
        xo_ref, = out_refs
    xr = xr_ref[0]
    xt = xt_ref[0]
    xc = xc_ref[0]
    sqj = jnp.zeros((1, N), F32)
    for c in range(F):
        row = xt[c:c + 1, :]
        sqj = sqj + row * row
    sqi = jnp.sum(xr * xr, axis=1, keepdims=True)
    dot = _dot(xr.astype(BF), xt.astype(BF))
    d = (sqi + sqj) - 2.0 * dot
    hi, mid, lo = _split3(xc)
    W16 = W_ref[...].astype(BF)
    brow = b_ref[...]
    iota = jax.lax.broadcasted_iota(jnp.int32, (R, N), 1)
    rowid = pl.program_id(1) * R + jax.lax.broadcasted_iota(
        jnp.int32, (R, 1), 0)
    acc = jnp.full((R, W_ref.shape[1]), -BIG, F32)
    dd = d
    dcols, icols = [], []
    for t in range(KNN):
        m = jnp.min(dd, axis=1, keepdims=True)
        a = jnp.min(jnp.where(dd == m, iota, N), axis=1, keepdims=True)
        oh = iota == a
        dd = jnp.where(oh, BIG, dd)
        if emit_knn:
            dcols.append(m)
            icols.append(a)
        oh16 = oh.astype(BF)
        xj = (_dot(oh16, hi) + _dot(oh16, mid)) + _dot(oh16, lo)
        cc = jnp.concatenate([xr, xj - xr], axis=1).astype(BF)
        msg = _dot(cc, W16) + brow
        msg = _leaky(msg)
        acc = jnp.maximum(acc, jnp.where(a != rowid, msg, -BIG))
    if emit_knn:
        dist_ref[0] = jnp.concatenate(dcols, axis=1)
        idx_ref[0] = jnp.concatenate(icols, axis=1)
    xo_ref[0] = acc


def _knn_edge(x, W, b, R, emit_knn):
    B, N, F = x.shape
    Fo = W.shape[1]
    xt = x.transpose(0, 2, 1)
    grid = (B, N // R)
    in_specs = [
        pl.BlockSpec((1, R, F), lambda bb, nb: (bb, nb, 0)),
        pl.BlockSpec((1, F, N), lambda bb, nb: (bb, 0, 0)),
        pl.BlockSpec((1, N, F), lambda bb, nb: (bb, 0, 0)),
        pl.BlockSpec((2 * F, Fo), lambda bb, nb: (0, 0)),
        pl.BlockSpec((1, Fo), lambda bb, nb: (0, 0)),
    ]
    if emit_knn:
        out_shape = [
            jax.ShapeDtypeStruct((B, N, KNN), F32),
            jax.ShapeDtypeStruct((B, N, KNN), jnp.int32),
            jax.ShapeDtypeStruct((B, N, Fo), F32),
        ]
        out_specs = [
            pl.BlockSpec((1, R, KNN), lambda bb, nb: (bb, nb, 0)),
            pl.BlockSpec((1, R, KNN), lambda bb, nb: (bb, nb, 0)),
            pl.BlockSpec((1, R, Fo), lambda bb, nb: (bb, nb, 0)),
        ]
    else:
        out_shape = [jax.ShapeDtypeStruct((B, N, Fo), F32)]
        out_specs = [pl.BlockSpec((1, R, Fo), lambda bb, nb: (bb, nb, 0))]
    f = pl.pallas_call(
        functools.partial(_knn_edge_body, R, N, F, emit_knn),
        grid=grid,
        in_specs=in_specs,
        out_specs=out_specs,
        out_shape=out_shape,
    )
    return f(x, xt, x, W, b.reshape(1, Fo))


def _hk(g16_k, ecol, l2W16, l2b, l1w, l1b, c3WT16, c3b):
    knnf = _dot(g16_k, l2W16) + l2b
    distf = ecol * l1w + l1b
    feat = jnp.concatenate([knnf, distf], axis=1)
    return _dot(feat.astype(BF), c3WT16) + c3b


def _stats_body(R, N, idx_ref, dist_ref, x2_ref, l2W_ref, l2b_ref, l1w_ref,
                l1b_ref, c3WT_ref, c3b_ref, g16_ref, bsum_ref, bss_ref,
                h_scr):
    x216 = x2_ref[0].astype(BF)
    l2W16 = l2W_ref[...].astype(BF)
    c3WT16 = c3WT_ref[...].astype(BF)
    l2b = l2b_ref[...]
    l1w = l1w_ref[...]
    l1b = l1b_ref[...]
    c3b = c3b_ref[...]
    iota = jax.lax.broadcasted_iota(jnp.int32, (R, N), 1)
    s = jnp.zeros((1, 128), F32)
    for k in range(KNN):
        oh = (iota == idx_ref[0][:, k:k + 1]).astype(BF)
        g_k = _dot(oh, x216)
        g16_ref[k] = g_k.astype(BF)
        ecol = jnp.exp(-dist_ref[0][:, k:k + 1])
        h = _hk(g_k.astype(BF), ecol, l2W16, l2b, l1w, l1b, c3WT16, c3b)
        h_scr[k * R:(k + 1) * R, :] = h
        s = s + jnp.sum(h, axis=0, keepdims=True)
    bsum_ref[0] = s
    bm = s / (KNN * R)
    ss = jnp.zeros((1, 128), F32)
    for k in range(KNN):
        hc = h_scr[k * R:(k + 1) * R, :] - bm
        ss = ss + jnp.sum(hc * hc, axis=0, keepdims=True)
    bss_ref[0] = ss


def _stats_pass(idx, dist, x2, l2W, l2b, l1W, l1b, c3WT, c3b, R):
    B, N, _ = x2.shape
    NB = N // R
    grid = (B, NB)
    f = pl.pallas_call(
        functools.partial(_stats_body, R, N),
        grid=grid,
        in_specs=[
            pl.BlockSpec((1, R, KNN), lambda bb, nb: (bb, nb, 0)),
            pl.BlockSpec((1, R, KNN), lambda bb, nb: (bb, nb, 0)),
            pl.BlockSpec((1, N, 48), lambda bb, nb: (bb, 0, 0)),
            pl.BlockSpec((48, 64), lambda bb, nb: (0, 0)),
            pl.BlockSpec((1, 64), lambda bb, nb: (0, 0)),
            pl.BlockSpec((1, 64), lambda bb, nb: (0, 0)),
            pl.BlockSpec((1, 64), lambda bb, nb: (0, 0)),
            pl.BlockSpec((128, 128), lambda bb, nb: (0, 0)),
            pl.BlockSpec((1, 128), lambda bb, nb: (0, 0)),
        ],
        out_specs=[
            pl.BlockSpec((KNN, R, 48),
                         lambda bb, nb, NB=NB: (0, bb * NB + nb, 0)),
            pl.BlockSpec((1, 1, 128), lambda bb, nb, NB=NB: (bb * NB + nb, 0, 0)),
            pl.BlockSpec((1, 1, 128), lambda bb, nb, NB=NB: (bb * NB + nb, 0, 0)),
        ],
        out_shape=[
            jax.ShapeDtypeStruct((KNN, B * N, 48), BF),
            jax.ShapeDtypeStruct((B * NB, 1, 128), F32),
            jax.ShapeDtypeStruct((B * NB, 1, 128), F32),
        ],
        scratch_shapes=[pltpu.VMEM((KNN * R, 128), F32)],
    )
    return f(idx, dist, x2, l2W, l2b.reshape(1, 64), l1W, l1b.reshape(1, 64),
             c3WT, c3b.reshape(1, 128))


def _score_body(R, N, g16_ref, idx_ref, dist_ref, x2_ref, l2W_ref, l2b_ref,
                l1w_ref, l1b_ref, c3WT_ref, c3b_ref, mu_ref, var_ref, g_ref,
                be_ref, W1_ref, b1_ref, z1_ref, bsum_ref, bss_ref):
    l2W16 = l2W_ref[...].astype(BF)
    c3WT16 = c3WT_ref[...].astype(BF)
    l2b = l2b_ref[...]
    l1w = l1w_ref[...]
    l1b = l1b_ref[...]
    c3b = c3b_ref[...]
    mu = mu_ref[...]
    den = jnp.sqrt(var_ref[...] + 1e-5)
    gg = g_ref[...]
    be = be_ref[...]
    scols = []
    for k in range(KNN):
        g_k = g16_ref[k]
        ecol = jnp.exp(-dist_ref[0][:, k:k + 1])
        h = _hk(g_k, ecol, l2W16, l2b, l1w, l1b, c3WT16, c3b)
        hh = _leaky((h - mu) / den * gg + be)
        scols.append(jnp.max(hh, axis=1, keepdims=True))
    score = jnp.concatenate(scols, axis=1)
    idxs = idx_ref[0]
    it = jax.lax.broadcasted_iota(jnp.int32, (R, KNN), 1)
    x216 = x2_ref[0].astype(BF)
    iota = jax.lax.broadcasted_iota(jnp.int32, (R, N), 1)
    W116 = W1_ref[...].astype(BF)
    z1 = jnp.zeros((R, 512), F32)
    s = score
    for t in range(NKEY):
        m = jnp.max(s, axis=1, keepdims=True)
        a = jnp.min(jnp.where(s == m, it, KNN), axis=1, keepdims=True)
        ohk = it == a
        s = jnp.where(ohk, -BIG, s)
        tp = jnp.sum(jnp.where(ohk, idxs, 0), axis=1, keepdims=True)
        oh = (iota == tp).astype(BF)
        keyf = _dot(oh, x216).astype(BF)
        z1 = z1 + _dot(keyf, W116[t * 48:(t + 1) * 48, :])
    z1 = z1 + b1_ref[...]
    z1_ref[...] = z1
    bs = jnp.sum(z1, axis=0, keepdims=True)
    bsum_ref[0] = bs
    zc = z1 - bs / R
    bss_ref[0] = jnp.sum(zc * zc, axis=0, keepdims=True)


def _score_pass(g16, idx, dist, x2, l2W, l2b, l1W, l1b, c3WT, c3b, mu, var,
                c3g, c3be, m1W1, m1b1, R):
    B, N, _ = x2.shape
    NB = N // R
    grid = (B, NB)
    f = pl.pallas_call(
        functools.partial(_score_body, R, N),
        grid=grid,
        in_specs=[
            pl.BlockSpec((KNN, R, 48), lambda bb, nb, NB=NB: (0, bb * NB + nb, 0)),
            pl.BlockSpec((1, R, KNN), lambda bb, nb: (bb, nb, 0)),
            pl.BlockSpec((1, R, KNN), lambda bb, nb: (bb, nb, 0)),
            pl.BlockSpec((1, N, 48), lambda bb, nb: (bb, 0, 0)),
            pl.BlockSpec((48, 64), lambda bb, nb: (0, 0)),
            pl.BlockSpec((1, 64), lambda bb, nb: (0, 0)),
            pl.BlockSpec((1, 64), lambda bb, nb: (0, 0)),
            pl.BlockSpec((1, 64), lambda bb, nb: (0, 0)),
            pl.BlockSpec((128, 128), lambda bb, nb: (0, 0)),
            pl.BlockSpec((1, 128), lambda bb, nb: (0, 0)),
            pl.BlockSpec((1, 128), lambda bb, nb: (0, 0)),
            pl.BlockSpec((1, 128), lambda bb, nb: (0, 0)),
            pl.BlockSpec((1, 128), lambda bb, nb: (0, 0)),
            pl.BlockSpec((1, 128), lambda bb, nb: (0, 0)),
            pl.BlockSpec((480, 512), lambda bb, nb: (0, 0)),
            pl.BlockSpec((1, 512), lambda bb, nb: (0, 0)),
        ],
        out_specs=[
            pl.BlockSpec((R, 512), lambda bb, nb, NB=NB: (bb * NB + nb, 0)),
            pl.BlockSpec((1, 1, 512), lambda bb, nb, NB=NB: (bb * NB + nb, 0, 0)),
            pl.BlockSpec((1, 1, 512), lambda bb, nb, NB=NB: (bb * NB + nb, 0, 0)),
        ],
        out_shape=[
            jax.ShapeDtypeStruct((B * N, 512), F32),
            jax.ShapeDtypeStruct((B * NB, 1, 512), F32),
            jax.ShapeDtypeStruct((B * NB, 1, 512), F32),
        ],
    )
    return f(g16, idx, dist, x2, l2W, l2b.reshape(1, 64), l1W,
             l1b.reshape(1, 64), c3WT, c3b.reshape(1, 128), mu, var,
             c3g.reshape(1, 128), c3be.reshape(1, 128), m1W1,
             m1b1.reshape(1, 512))


def _mlp2_body(R, z1_ref, mu_ref, var_ref, g_ref, be_ref, W2_ref, b2_ref,
               z2_ref, bsum_ref, bss_ref):
    den = jnp.sqrt(var_ref[...] + 1e-5)
    a1 = _leaky((z1_ref[...] - mu_ref[...]) / den * g_ref[...] + be_ref[...])
    z2 = _dot(a1.astype(BF), W2_ref[...].astype(BF)) + b2_ref[...]
    z2_ref[...] = z2
    bs = jnp.sum(z2, axis=0, keepdims=True)
    bsum_ref[0] = bs
    zc = z2 - bs / R
    bss_ref[0] = jnp.sum(zc * zc, axis=0, keepdims=True)


def _mlp2_pass(z1, mu1, var1, g1, be1, m1W2, m1b2, R):
    M = z1.shape[0]
    NB = M // R
    f = pl.pallas_call(
        functools.partial(_mlp2_body, R),
        grid=(NB,),
        in_specs=[
            pl.BlockSpec((R, 512), lambda i: (i, 0)),
            pl.BlockSpec((1, 512), lambda i: (0, 0)),
            pl.BlockSpec((1, 512), lambda i: (0, 0)),
            pl.BlockSpec((1, 512), lambda i: (0, 0)),
            pl.BlockSpec((1, 512), lambda i: (0, 0)),
            pl.BlockSpec((512, 512), lambda i: (0, 0)),
            pl.BlockSpec((1, 512), lambda i: (0, 0)),
        ],
        out_specs=[
            pl.BlockSpec((R, 512), lambda i: (i, 0)),
            pl.BlockSpec((1, 1, 512), lambda i: (i, 0, 0)),
            pl.BlockSpec((1, 1, 512), lambda i: (i, 0, 0)),
        ],
        out_shape=[
            jax.ShapeDtypeStruct((M, 512), F32),
            jax.ShapeDtypeStruct((NB, 1, 512), F32),
            jax.ShapeDtypeStruct((NB, 1, 512), F32),
        ],
    )
    return f(z1, mu1, var1, g1.reshape(1, 512), be1.reshape(1, 512), m1W2,
             m1b2.reshape(1, 512))


def _final_body(z2_ref, mu_ref, var_ref, g_ref, be_ref, o_ref):
    den = jnp.sqrt(var_ref[...] + 1e-5)
    o_ref[...] = _leaky(
        (z2_ref[...] - mu_ref[...]) / den * g_ref[...] + be_ref[...])


def _final_pass(z2, mu2, var2, g2, be2, R):
    M = z2.shape[0]
    NB = M // R
    f = pl.pallas_call(
        _final_body,
        grid=(NB,),
        in_specs=[
            pl.BlockSpec((R, 512), lambda i: (i, 0)),
            pl.BlockSpec((1, 512), lambda i: (0, 0)),
            pl.BlockSpec((1, 512), lambda i: (0, 0)),
            pl.BlockSpec((1, 512), lambda i: (0, 0)),
            pl.BlockSpec((1, 512), lambda i: (0, 0)),
        ],
        out_specs=[pl.BlockSpec((R, 512), lambda i: (i, 0))],
        out_shape=[jax.ShapeDtypeStruct((M, 512), F32)],
    )
    return f(z2, mu2, var2, g2.reshape(1, 512), be2.reshape(1, 512))[0]


def _combine_stats(bsum, bss, m, cnt):
    bsum = bsum.reshape(bsum.shape[0], -1)
    bss = bss.reshape(bss.shape[0], -1)
    mu = jnp.sum(bsum, axis=0, keepdims=True) / cnt
    bm = bsum / m
    var = (jnp.sum(bss, axis=0, keepdims=True)
           + m * jnp.sum((bm - mu) ** 2, axis=0, keepdims=True)) / cnt
    return mu, var


def kernel(x, W1, b1, W2, b2, l1W, l1b, l2W, l2b, c3W, c3b, c3g, c3be,
           m1W1, m1b1, m1g1, m1be1, m1W2, m1b2, m1g2, m1be2):
    B, N, _ = x.shape
    R = 128
    dist, idx, x1 = _knn_edge(x, W1, b1, R, emit_knn=True)
    x2, = _knn_edge(x1, W2, b2, R, emit_knn=False)

    c3WT = c3W.T
    g16, bsum, bss = _stats_pass(idx, dist, x2, l2W, l2b, l1W, l1b, c3WT,
                                 c3b, R)
    mu, var = _combine_stats(bsum, bss, KNN * R, B * N * KNN)

    z1, bsum1, bss1 = _score_pass(g16, idx, dist, x2, l2W, l2b, l1W, l1b,
                                  c3WT, c3b, mu, var, c3g, c3be, m1W1,
                                  m1b1, R)
    mu1, var1 = _combine_stats(bsum1, bss1, R, B * N)

    Rz = 512
    z2, bsum2, bss2 = _mlp2_pass(z1, mu1, var1, m1g1, m1be1, m1W2, m1b2, Rz)
    mu2, var2 = _combine_stats(bsum2, bss2, Rz, B * N)

    out = _final_pass(z2, mu2, var2, m1g2, m1be2, Rz)
    return out.reshape(B, N, 512)


# speedup vs baseline: 4.5212x; 1.2844x over previous
import functools

import jax
import jax.numpy as jnp
from jax.experimental import pallas as pl
from jax.experimental.pallas import tpu as pltpu

KNN = 33
NKEY = 10
BIG = 3e38
F32 = jnp.float32
BF = jnp.bfloat16


def _split3(x):
    hi = x.astype(BF)
    r1 = x - hi.astype(F32)
    mid = r1.astype(BF)
    lo = (r1 - mid.astype(F32)).astype(BF)
    return hi, mid, lo


def _leaky(x):
    return jnp.where(x >= 0, x, 0.2 * x)


def _dot(a, b):
    return jnp.dot(a, b, preferred_element_type=F32)


def _knn_edge_body(R, N, F, emit_knn, xr_ref, xt_ref, xc_ref, W_ref, b_ref,
                   *out_refs):
    if emit_knn:
        dist_ref, idx_ref, xo_ref = out_refs
    else:
        xo_ref, = out_refs
    xr = xr_ref[0]
    xt = xt_ref[0]
    xc = xc_ref[0]
    sqj = jnp.zeros((1, N), F32)
    for c in range(F):
        row = xt[c:c + 1, :]
        sqj = sqj + row * row
    sqi = jnp.sum(xr * xr, axis=1, keepdims=True)
    dot = _dot(xr.astype(BF), xt.astype(BF))
    d = (sqi + sqj) - 2.0 * dot
    hi, mid, lo = _split3(xc)
    tbl = jnp.concatenate([hi, mid, lo], axis=1)
    W16 = W_ref[...].astype(BF)
    brow = b_ref[...]
    iota = jax.lax.broadcasted_iota(jnp.int32, (R, N), 1)
    rowid = pl.program_id(1) * R + jax.lax.broadcasted_iota(
        jnp.int32, (R, 1), 0)
    acc = jnp.full((R, W_ref.shape[1]), -BIG, F32)
    dd = d
    dcols, icols = [], []
    for t in range(KNN):
        m = jnp.min(dd, axis=1, keepdims=True)
        a = jnp.min(jnp.where(dd == m, iota, N), axis=1, keepdims=True)
        oh = iota == a
        dd = jnp.where(oh, BIG, dd)
        if emit_knn:
            dcols.append(m)
            icols.append(a)
        oh16 = oh.astype(BF)
        parts = _dot(oh16, tbl)
        xj = (parts[:, :F] + parts[:, F:2 * F]) + parts[:, 2 * F:]
        cc = jnp.concatenate([xr, xj - xr], axis=1).astype(BF)
        msg = _dot(cc, W16) + brow
        msg = _leaky(msg)
        acc = jnp.maximum(acc, jnp.where(a != rowid, msg, -BIG))
    if emit_knn:
        dist_ref[0] = jnp.concatenate(dcols, axis=1)
        idx_ref[0] = jnp.concatenate(icols, axis=1)
    xo_ref[0] = acc


def _knn_edge(x, W, b, R, emit_knn):
    B, N, F = x.shape
    Fo = W.shape[1]
    xt = x.transpose(0, 2, 1)
    grid = (B, N // R)
    in_specs = [
        pl.BlockSpec((1, R, F), lambda bb, nb: (bb, nb, 0)),
        pl.BlockSpec((1, F, N), lambda bb, nb: (bb, 0, 0)),
        pl.BlockSpec((1, N, F), lambda bb, nb: (bb, 0, 0)),
        pl.BlockSpec((2 * F, Fo), lambda bb, nb: (0, 0)),
        pl.BlockSpec((1, Fo), lambda bb, nb: (0, 0)),
    ]
    if emit_knn:
        out_shape = [
            jax.ShapeDtypeStruct((B, N, KNN), F32),
            jax.ShapeDtypeStruct((B, N, KNN), jnp.int32),
            jax.ShapeDtypeStruct((B, N, Fo), F32),
        ]
        out_specs = [
            pl.BlockSpec((1, R, KNN), lambda bb, nb: (bb, nb, 0)),
            pl.BlockSpec((1, R, KNN), lambda bb, nb: (bb, nb, 0)),
            pl.BlockSpec((1, R, Fo), lambda bb, nb: (bb, nb, 0)),
        ]
    else:
        out_shape = [jax.ShapeDtypeStruct((B, N, Fo), F32)]
        out_specs = [pl.BlockSpec((1, R, Fo), lambda bb, nb: (bb, nb, 0))]
    f = pl.pallas_call(
        functools.partial(_knn_edge_body, R, N, F, emit_knn),
        grid=grid,
        in_specs=in_specs,
        out_specs=out_specs,
        out_shape=out_shape,
    )
    return f(x, xt, x, W, b.reshape(1, Fo))


def _hk(g16_k, ecol, l2W16, l2b, l1w, l1b, c3WT16, c3b):
    knnf = _dot(g16_k, l2W16) + l2b
    distf = ecol * l1w + l1b
    feat = jnp.concatenate([knnf, distf], axis=1)
    return _dot(feat.astype(BF), c3WT16) + c3b


def _stats_body(R, N, idx_ref, dist_ref, x2_ref, l2W_ref, l2b_ref, l1w_ref,
                l1b_ref, c3WT_ref, c3b_ref, g16_ref, bsum_ref, bss_ref,
                h_scr):
    x216 = x2_ref[0].astype(BF)
    l2W16 = l2W_ref[...].astype(BF)
    c3WT16 = c3WT_ref[...].astype(BF)
    l2b = l2b_ref[...]
    l1w = l1w_ref[...]
    l1b = l1b_ref[...]
    c3b = c3b_ref[...]
    iota = jax.lax.broadcasted_iota(jnp.int32, (R, N), 1)
    s = jnp.zeros((1, 128), F32)
    for k in range(KNN):
        oh = (iota == idx_ref[0][:, k:k + 1]).astype(BF)
        g_k = _dot(oh, x216)
        g16_ref[k] = g_k.astype(BF)
        ecol = jnp.exp(-dist_ref[0][:, k:k + 1])
        h = _hk(g_k.astype(BF), ecol, l2W16, l2b, l1w, l1b, c3WT16, c3b)
        h_scr[k * R:(k + 1) * R, :] = h
        s = s + jnp.sum(h, axis=0, keepdims=True)
    bsum_ref[0] = s
    bm = s / (KNN * R)
    ss = jnp.zeros((1, 128), F32)
    for k in range(KNN):
        hc = h_scr[k * R:(k + 1) * R, :] - bm
        ss = ss + jnp.sum(hc * hc, axis=0, keepdims=True)
    bss_ref[0] = ss


def _stats_pass(idx, dist, x2, l2W, l2b, l1W, l1b, c3WT, c3b, R):
    B, N, _ = x2.shape
    NB = N // R
    grid = (B, NB)
    f = pl.pallas_call(
        functools.partial(_stats_body, R, N),
        grid=grid,
        in_specs=[
            pl.BlockSpec((1, R, KNN), lambda bb, nb: (bb, nb, 0)),
            pl.BlockSpec((1, R, KNN), lambda bb, nb: (bb, nb, 0)),
            pl.BlockSpec((1, N, 48), lambda bb, nb: (bb, 0, 0)),
            pl.BlockSpec((48, 64), lambda bb, nb: (0, 0)),
            pl.BlockSpec((1, 64), lambda bb, nb: (0, 0)),
            pl.BlockSpec((1, 64), lambda bb, nb: (0, 0)),
            pl.BlockSpec((1, 64), lambda bb, nb: (0, 0)),
            pl.BlockSpec((128, 128), lambda bb, nb: (0, 0)),
            pl.BlockSpec((1, 128), lambda bb, nb: (0, 0)),
        ],
        out_specs=[
            pl.BlockSpec((KNN, R, 48),
                         lambda bb, nb, NB=NB: (0, bb * NB + nb, 0)),
            pl.BlockSpec((1, 1, 128), lambda bb, nb, NB=NB: (bb * NB + nb, 0, 0)),
            pl.BlockSpec((1, 1, 128), lambda bb, nb, NB=NB: (bb * NB + nb, 0, 0)),
        ],
        out_shape=[
            jax.ShapeDtypeStruct((KNN, B * N, 48), BF),
            jax.ShapeDtypeStruct((B * NB, 1, 128), F32),
            jax.ShapeDtypeStruct((B * NB, 1, 128), F32),
        ],
        scratch_shapes=[pltpu.VMEM((KNN * R, 128), F32)],
    )
    return f(idx, dist, x2, l2W, l2b.reshape(1, 64), l1W, l1b.reshape(1, 64),
             c3WT, c3b.reshape(1, 128))


def _score_body(R, N, g16_ref, idx_ref, dist_ref, x2_ref, l2W_ref, l2b_ref,
                l1w_ref, l1b_ref, c3WT_ref, c3b_ref, mu_ref, var_ref, g_ref,
                be_ref, W1_ref, b1_ref, z1_ref, bsum_ref, bss_ref):
    l2W16 = l2W_ref[...].astype(BF)
    c3WT16 = c3WT_ref[...].astype(BF)
    l2b = l2b_ref[...]
    l1w = l1w_ref[...]
    l1b = l1b_ref[...]
    c3b = c3b_ref[...]
    mu = mu_ref[...]
    den = jnp.sqrt(var_ref[...] + 1e-5)
    gg = g_ref[...]
    be = be_ref[...]
    scols = []
    for k in range(KNN):
        g_k = g16_ref[k]
        ecol = jnp.exp(-dist_ref[0][:, k:k + 1])
        h = _hk(g_k, ecol, l2W16, l2b, l1w, l1b, c3WT16, c3b)
        hh = _leaky((h - mu) / den * gg + be)
        scols.append(jnp.max(hh, axis=1, keepdims=True))
    score = jnp.concatenate(scols, axis=1)
    idxs = idx_ref[0]
    it = jax.lax.broadcasted_iota(jnp.int32, (R, KNN), 1)
    x216 = x2_ref[0].astype(BF)
    iota = jax.lax.broadcasted_iota(jnp.int32, (R, N), 1)
    W116 = W1_ref[...].astype(BF)
    z1 = jnp.zeros((R, 512), F32)
    s = score
    for t in range(NKEY):
        m = jnp.max(s, axis=1, keepdims=True)
        a = jnp.min(jnp.where(s == m, it, KNN), axis=1, keepdims=True)
        ohk = it == a
        s = jnp.where(ohk, -BIG, s)
        tp = jnp.sum(jnp.where(ohk, idxs, 0), axis=1, keepdims=True)
        oh = (iota == tp).astype(BF)
        keyf = _dot(oh, x216).astype(BF)
        z1 = z1 + _dot(keyf, W116[t * 48:(t + 1) * 48, :])
    z1 = z1 + b1_ref[...]
    z1_ref[...] = z1
    bs = jnp.sum(z1, axis=0, keepdims=True)
    bsum_ref[0] = bs
    zc = z1 - bs / R
    bss_ref[0] = jnp.sum(zc * zc, axis=0, keepdims=True)


def _score_pass(g16, idx, dist, x2, l2W, l2b, l1W, l1b, c3WT, c3b, mu, var,
                c3g, c3be, m1W1, m1b1, R):
    B, N, _ = x2.shape
    NB = N // R
    grid = (B, NB)
    f = pl.pallas_call(
        functools.partial(_score_body, R, N),
        grid=grid,
        in_specs=[
            pl.BlockSpec((KNN, R, 48), lambda bb, nb, NB=NB: (0, bb * NB + nb, 0)),
            pl.BlockSpec((1, R, KNN), lambda bb, nb: (bb, nb, 0)),
            pl.BlockSpec((1, R, KNN), lambda bb, nb: (bb, nb, 0)),
            pl.BlockSpec((1, N, 48), lambda bb, nb: (bb, 0, 0)),
            pl.BlockSpec((48, 64), lambda bb, nb: (0, 0)),
            pl.BlockSpec((1, 64), lambda bb, nb: (0, 0)),
            pl.BlockSpec((1, 64), lambda bb, nb: (0, 0)),
            pl.BlockSpec((1, 64), lambda bb, nb: (0, 0)),
            pl.BlockSpec((128, 128), lambda bb, nb: (0, 0)),
            pl.BlockSpec((1, 128), lambda bb, nb: (0, 0)),
            pl.BlockSpec((1, 128), lambda bb, nb: (0, 0)),
            pl.BlockSpec((1, 128), lambda bb, nb: (0, 0)),
            pl.BlockSpec((1, 128), lambda bb, nb: (0, 0)),
            pl.BlockSpec((1, 128), lambda bb, nb: (0, 0)),
            pl.BlockSpec((480, 512), lambda bb, nb: (0, 0)),
            pl.BlockSpec((1, 512), lambda bb, nb: (0, 0)),
        ],
        out_specs=[
            pl.BlockSpec((R, 512), lambda bb, nb, NB=NB: (bb * NB + nb, 0)),
            pl.BlockSpec((1, 1, 512), lambda bb, nb, NB=NB: (bb * NB + nb, 0, 0)),
            pl.BlockSpec((1, 1, 512), lambda bb, nb, NB=NB: (bb * NB + nb, 0, 0)),
        ],
        out_shape=[
            jax.ShapeDtypeStruct((B * N, 512), F32),
            jax.ShapeDtypeStruct((B * NB, 1, 512), F32),
            jax.ShapeDtypeStruct((B * NB, 1, 512), F32),
        ],
    )
    return f(g16, idx, dist, x2, l2W, l2b.reshape(1, 64), l1W,
             l1b.reshape(1, 64), c3WT, c3b.reshape(1, 128), mu, var,
             c3g.reshape(1, 128), c3be.reshape(1, 128), m1W1,
             m1b1.reshape(1, 512))


def _mlp2_body(R, z1_ref, mu_ref, var_ref, g_ref, be_ref, W2_ref, b2_ref,
               z2_ref, bsum_ref, bss_ref):
    den = jnp.sqrt(var_ref[...] + 1e-5)
    a1 = _leaky((z1_ref[...] - mu_ref[...]) / den * g_ref[...] + be_ref[...])
    z2 = _dot(a1.astype(BF), W2_ref[...].astype(BF)) + b2_ref[...]
    z2_ref[...] = z2
    bs = jnp.sum(z2, axis=0, keepdims=True)
    bsum_ref[0] = bs
    zc = z2 - bs / R
    bss_ref[0] = jnp.sum(zc * zc, axis=0, keepdims=True)


def _mlp2_pass(z1, mu1, var1, g1, be1, m1W2, m1b2, R):
    M = z1.shape[0]
    NB = M // R
    f = pl.pallas_call(
        functools.partial(_mlp2_body, R),
        grid=(NB,),
        in_specs=[
            pl.BlockSpec((R, 512), lambda i: (i, 0)),
            pl.BlockSpec((1, 512), lambda i: (0, 0)),
            pl.BlockSpec((1, 512), lambda i: (0, 0)),
            pl.BlockSpec((1, 512), lambda i: (0, 0)),
            pl.BlockSpec((1, 512), lambda i: (0, 0)),
            pl.BlockSpec((512, 512), lambda i: (0, 0)),
            pl.BlockSpec((1, 512), lambda i: (0, 0)),
        ],
        out_specs=[
            pl.BlockSpec((R, 512), lambda i: (i, 0)),
            pl.BlockSpec((1, 1, 512), lambda i: (i, 0, 0)),
            pl.BlockSpec((1, 1, 512), lambda i: (i, 0, 0)),
        ],
        out_shape=[
            jax.ShapeDtypeStruct((M, 512), F32),
            jax.ShapeDtypeStruct((NB, 1, 512), F32),
            jax.ShapeDtypeStruct((NB, 1, 512), F32),
        ],
    )
    return f(z1, mu1, var1, g1.reshape(1, 512), be1.reshape(1, 512), m1W2,
             m1b2.reshape(1, 512))


def _final_body(z2_ref, mu_ref, var_ref, g_ref, be_ref, o_ref):
    den = jnp.sqrt(var_ref[...] + 1e-5)
    o_ref[...] = _leaky(
        (z2_ref[...] - mu_ref[...]) / den * g_ref[...] + be_ref[...])


def _final_pass(z2, mu2, var2, g2, be2, R):
    M = z2.shape[0]
    NB = M // R
    f = pl.pallas_call(
        _final_body,
        grid=(NB,),
        in_specs=[
            pl.BlockSpec((R, 512), lambda i: (i, 0)),
            pl.BlockSpec((1, 512), lambda i: (0, 0)),
            pl.BlockSpec((1, 512), lambda i: (0, 0)),
            pl.BlockSpec((1, 512), lambda i: (0, 0)),
            pl.BlockSpec((1, 512), lambda i: (0, 0)),
        ],
        out_specs=[pl.BlockSpec((R, 512), lambda i: (i, 0))],
        out_shape=[jax.ShapeDtypeStruct((M, 512), F32)],
    )
    return f(z2, mu2, var2, g2.reshape(1, 512), be2.reshape(1, 512))[0]


def _combine_stats(bsum, bss, m, cnt):
    bsum = bsum.reshape(bsum.shape[0], -1)
    bss = bss.reshape(bss.shape[0], -1)
    mu = jnp.sum(bsum, axis=0, keepdims=True) / cnt
    bm = bsum / m
    var = (jnp.sum(bss, axis=0, keepdims=True)
           + m * jnp.sum((bm - mu) ** 2, axis=0, keepdims=True)) / cnt
    return mu, var


def kernel(x, W1, b1, W2, b2, l1W, l1b, l2W, l2b, c3W, c3b, c3g, c3be,
           m1W1, m1b1, m1g1, m1be1, m1W2, m1b2, m1g2, m1be2):
    B, N, _ = x.shape
    R = 128
    dist, idx, x1 = _knn_edge(x, W1, b1, R, emit_knn=True)
    x2, = _knn_edge(x1, W2, b2, R, emit_knn=False)

    c3WT = c3W.T
    g16, bsum, bss = _stats_pass(idx, dist, x2, l2W, l2b, l1W, l1b, c3WT,
                                 c3b, R)
    mu, var = _combine_stats(bsum, bss, KNN * R, B * N * KNN)

    z1, bsum1, bss1 = _score_pass(g16, idx, dist, x2, l2W, l2b, l1W, l1b,
                                  c3WT, c3b, mu, var, c3g, c3be, m1W1,
                                  m1b1, R)
    mu1, var1 = _combine_stats(bsum1, bss1, R, B * N)

    Rz = 512
    z2, bsum2, bss2 = _mlp2_pass(z1, mu1, var1, m1g1, m1be1, m1W2, m1b2, Rz)
    mu2, var2 = _combine_stats(bsum2, bss2, Rz, B * N)

    out = _final_pass(z2, mu2, var2, m1g2, m1be2, Rz)
    return out.reshape(B, N, 512)


# speedup vs baseline: 4.6641x; 1.0316x over previous
import functools

import jax
import jax.numpy as jnp
from jax.experimental import pallas as pl
from jax.experimental.pallas import tpu as pltpu

KNN = 33
NKEY = 10
BIG = 3e38
F32 = jnp.float32
BF = jnp.bfloat16


def _split3(x):
    hi = x.astype(BF)
    r1 = x - hi.astype(F32)
    mid = r1.astype(BF)
    lo = (r1 - mid.astype(F32)).astype(BF)
    return hi, mid, lo


def _leaky(x):
    return jnp.where(x >= 0, x, 0.2 * x)


def _dot(a, b):
    return jnp.dot(a, b, preferred_element_type=F32)


def _knn_edge_body(R, N, F, emit_knn, xr_ref, xt_ref, xc_ref, W_ref, b_ref,
                   *out_refs):
    if emit_knn:
        dist_ref, idx_ref, xo_ref = out_refs
    else:
        xo_ref, = out_refs
    xr = xr_ref[0]
    xt = xt_ref[0]
    xc = xc_ref[0]
    sqj = jnp.zeros((1, N), F32)
    for c in range(F):
        row = xt[c:c + 1, :]
        sqj = sqj + row * row
    sqi = jnp.sum(xr * xr, axis=1, keepdims=True)
    dot = _dot(xr.astype(BF), xt.astype(BF))
    d = (sqi + sqj) - 2.0 * dot
    hi, mid, lo = _split3(xc)
    tbl = jnp.concatenate([hi, mid, lo], axis=1)
    W16 = W_ref[...].astype(BF)
    brow = b_ref[...]
    iota = jax.lax.broadcasted_iota(jnp.int32, (R, N), 1)
    rowid = pl.program_id(1) * R + jax.lax.broadcasted_iota(
        jnp.int32, (R, 1), 0)
    acc = jnp.full((R, W_ref.shape[1]), -BIG, F32)
    dd = d
    dcols, icols = [], []
    for t in range(KNN):
        m = jnp.min(dd, axis=1, keepdims=True)
        a = jnp.min(jnp.where(dd == m, iota, N), axis=1, keepdims=True)
        oh = iota == a
        dd = jnp.where(oh, BIG, dd)
        if emit_knn:
            dcols.append(m)
            icols.append(a)
        oh16 = oh.astype(BF)
        parts = _dot(oh16, tbl)
        xj = (parts[:, :F] + parts[:, F:2 * F]) + parts[:, 2 * F:]
        cc = jnp.concatenate([xr, xj - xr], axis=1).astype(BF)
        msg = _dot(cc, W16) + brow
        msg = _leaky(msg)
        acc = jnp.maximum(acc, jnp.where(a != rowid, msg, -BIG))
    if emit_knn:
        dist_ref[0] = jnp.concatenate(dcols, axis=1)
        idx_ref[0] = jnp.concatenate(icols, axis=1)
    xo_ref[0] = acc


def _knn_edge(x, W, b, R, emit_knn):
    B, N, F = x.shape
    Fo = W.shape[1]
    xt = x.transpose(0, 2, 1)
    grid = (B, N // R)
    in_specs = [
        pl.BlockSpec((1, R, F), lambda bb, nb: (bb, nb, 0)),
        pl.BlockSpec((1, F, N), lambda bb, nb: (bb, 0, 0)),
        pl.BlockSpec((1, N, F), lambda bb, nb: (bb, 0, 0)),
        pl.BlockSpec((2 * F, Fo), lambda bb, nb: (0, 0)),
        pl.BlockSpec((1, Fo), lambda bb, nb: (0, 0)),
    ]
    if emit_knn:
        out_shape = [
            jax.ShapeDtypeStruct((B, N, KNN), F32),
            jax.ShapeDtypeStruct((B, N, KNN), jnp.int32),
            jax.ShapeDtypeStruct((B, N, Fo), F32),
        ]
        out_specs = [
            pl.BlockSpec((1, R, KNN), lambda bb, nb: (bb, nb, 0)),
            pl.BlockSpec((1, R, KNN), lambda bb, nb: (bb, nb, 0)),
            pl.BlockSpec((1, R, Fo), lambda bb, nb: (bb, nb, 0)),
        ]
    else:
        out_shape = [jax.ShapeDtypeStruct((B, N, Fo), F32)]
        out_specs = [pl.BlockSpec((1, R, Fo), lambda bb, nb: (bb, nb, 0))]
    f = pl.pallas_call(
        functools.partial(_knn_edge_body, R, N, F, emit_knn),
        grid=grid,
        in_specs=in_specs,
        out_specs=out_specs,
        out_shape=out_shape,
    )
    return f(x, xt, x, W, b.reshape(1, Fo))


def _hk(g16_k, ecol, l2W16, l2b, l1w, l1b, c3WT16, c3b):
    knnf = _dot(g16_k, l2W16) + l2b
    distf = ecol * l1w + l1b
    feat = jnp.concatenate([knnf, distf], axis=1)
    return _dot(feat.astype(BF), c3WT16) + c3b


def _stats_body(R, N, idx_ref, dist_ref, x2_ref, l2W_ref, l2b_ref, l1w_ref,
                l1b_ref, c3WT_ref, c3b_ref, g16_ref, bsum_ref, bss_ref,
                h_scr):
    x216 = x2_ref[0].astype(BF)
    l2W16 = l2W_ref[...].astype(BF)
    c3WT16 = c3WT_ref[...].astype(BF)
    l2b = l2b_ref[...]
    l1w = l1w_ref[...]
    l1b = l1b_ref[...]
    c3b = c3b_ref[...]
    iota = jax.lax.broadcasted_iota(jnp.int32, (R, N), 1)
    s = jnp.zeros((1, 128), F32)
    for k in range(KNN):
        oh = (iota == idx_ref[0][:, k:k + 1]).astype(BF)
        g_k = _dot(oh, x216)
        g16_ref[k] = g_k.astype(BF)
        ecol = jnp.exp(-dist_ref[0][:, k:k + 1])
        h = _hk(g_k.astype(BF), ecol, l2W16, l2b, l1w, l1b, c3WT16, c3b)
        h_scr[k * R:(k + 1) * R, :] = h
        s = s + jnp.sum(h, axis=0, keepdims=True)
    bsum_ref[0] = s
    bm = s / (KNN * R)
    ss = jnp.zeros((1, 128), F32)
    for k in range(KNN):
        hc = h_scr[k * R:(k + 1) * R, :] - bm
        ss = ss + jnp.sum(hc * hc, axis=0, keepdims=True)
    bss_ref[0] = ss


def _stats_pass(idx, dist, x2, l2W, l2b, l1W, l1b, c3WT, c3b, R):
    B, N, _ = x2.shape
    NB = N // R
    grid = (B, NB)
    f = pl.pallas_call(
        functools.partial(_stats_body, R, N),
        grid=grid,
        in_specs=[
            pl.BlockSpec((1, R, KNN), lambda bb, nb: (bb, nb, 0)),
            pl.BlockSpec((1, R, KNN), lambda bb, nb: (bb, nb, 0)),
            pl.BlockSpec((1, N, 48), lambda bb, nb: (bb, 0, 0)),
            pl.BlockSpec((48, 64), lambda bb, nb: (0, 0)),
            pl.BlockSpec((1, 64), lambda bb, nb: (0, 0)),
            pl.BlockSpec((1, 64), lambda bb, nb: (0, 0)),
            pl.BlockSpec((1, 64), lambda bb, nb: (0, 0)),
            pl.BlockSpec((128, 128), lambda bb, nb: (0, 0)),
            pl.BlockSpec((1, 128), lambda bb, nb: (0, 0)),
        ],
        out_specs=[
            pl.BlockSpec((KNN, R, 48),
                         lambda bb, nb, NB=NB: (0, bb * NB + nb, 0)),
            pl.BlockSpec((1, 1, 128), lambda bb, nb, NB=NB: (bb * NB + nb, 0, 0)),
            pl.BlockSpec((1, 1, 128), lambda bb, nb, NB=NB: (bb * NB + nb, 0, 0)),
        ],
        out_shape=[
            jax.ShapeDtypeStruct((KNN, B * N, 48), BF),
            jax.ShapeDtypeStruct((B * NB, 1, 128), F32),
            jax.ShapeDtypeStruct((B * NB, 1, 128), F32),
        ],
        scratch_shapes=[pltpu.VMEM((KNN * R, 128), F32)],
    )
    return f(idx, dist, x2, l2W, l2b.reshape(1, 64), l1W, l1b.reshape(1, 64),
             c3WT, c3b.reshape(1, 128))


def _score_body(R, N, g16_ref, idx_ref, dist_ref, x2_ref, l2W_ref, l2b_ref,
                l1w_ref, l1b_ref, c3WT_ref, c3b_ref, mu_ref, var_ref, g_ref,
                be_ref, W1_ref, b1_ref, z1_ref, bsum_ref, bss_ref):
    l2W16 = l2W_ref[...].astype(BF)
    c3WT16 = c3WT_ref[...].astype(BF)
    l2b = l2b_ref[...]
    l1w = l1w_ref[...]
    l1b = l1b_ref[...]
    c3b = c3b_ref[...]
    mu = mu_ref[...]
    den = jnp.sqrt(var_ref[...] + 1e-5)
    gg = g_ref[...]
    be = be_ref[...]
    scols = []
    for k in range(KNN):
        g_k = g16_ref[k]
        ecol = jnp.exp(-dist_ref[0][:, k:k + 1])
        h = _hk(g_k, ecol, l2W16, l2b, l1w, l1b, c3WT16, c3b)
        hh = _leaky((h - mu) / den * gg + be)
        scols.append(jnp.max(hh, axis=1, keepdims=True))
    score = jnp.concatenate(scols, axis=1)
    idxs = idx_ref[0]
    it = jax.lax.broadcasted_iota(jnp.int32, (R, KNN), 1)
    x216 = x2_ref[0].astype(BF)
    iota = jax.lax.broadcasted_iota(jnp.int32, (R, N), 1)
    W116 = W1_ref[...].astype(BF)
    z1 = jnp.zeros((R, 512), F32)
    s = score
    for t in range(NKEY):
        m = jnp.max(s, axis=1, keepdims=True)
        a = jnp.min(jnp.where(s == m, it, KNN), axis=1, keepdims=True)
        ohk = it == a
        s = jnp.where(ohk, -BIG, s)
        tp = jnp.sum(jnp.where(ohk, idxs, 0), axis=1, keepdims=True)
        oh = (iota == tp).astype(BF)
        keyf = _dot(oh, x216).astype(BF)
        z1 = z1 + _dot(keyf, W116[t * 48:(t + 1) * 48, :])
    z1 = z1 + b1_ref[...]
    z1_ref[...] = z1
    bs = jnp.sum(z1, axis=0, keepdims=True)
    bsum_ref[0] = bs
    zc = z1 - bs / R
    bss_ref[0] = jnp.sum(zc * zc, axis=0, keepdims=True)


def _score_pass(g16, idx, dist, x2, l2W, l2b, l1W, l1b, c3WT, c3b, mu, var,
                c3g, c3be, m1W1, m1b1, R):
    B, N, _ = x2.shape
    NB = N // R
    grid = (B, NB)
    f = pl.pallas_call(
        functools.partial(_score_body, R, N),
        grid=grid,
        in_specs=[
            pl.BlockSpec((KNN, R, 48), lambda bb, nb, NB=NB: (0, bb * NB + nb, 0)),
            pl.BlockSpec((1, R, KNN), lambda bb, nb: (bb, nb, 0)),
            pl.BlockSpec((1, R, KNN), lambda bb, nb: (bb, nb, 0)),
            pl.BlockSpec((1, N, 48), lambda bb, nb: (bb, 0, 0)),
            pl.BlockSpec((48, 64), lambda bb, nb: (0, 0)),
            pl.BlockSpec((1, 64), lambda bb, nb: (0, 0)),
            pl.BlockSpec((1, 64), lambda bb, nb: (0, 0)),
            pl.BlockSpec((1, 64), lambda bb, nb: (0, 0)),
            pl.BlockSpec((128, 128), lambda bb, nb: (0, 0)),
            pl.BlockSpec((1, 128), lambda bb, nb: (0, 0)),
            pl.BlockSpec((1, 128), lambda bb, nb: (0, 0)),
            pl.BlockSpec((1, 128), lambda bb, nb: (0, 0)),
            pl.BlockSpec((1, 128), lambda bb, nb: (0, 0)),
            pl.BlockSpec((1, 128), lambda bb, nb: (0, 0)),
            pl.BlockSpec((480, 512), lambda bb, nb: (0, 0)),
            pl.BlockSpec((1, 512), lambda bb, nb: (0, 0)),
        ],
        out_specs=[
            pl.BlockSpec((R, 512), lambda bb, nb, NB=NB: (bb * NB + nb, 0)),
            pl.BlockSpec((1, 1, 512), lambda bb, nb, NB=NB: (bb * NB + nb, 0, 0)),
            pl.BlockSpec((1, 1, 512), lambda bb, nb, NB=NB: (bb * NB + nb, 0, 0)),
        ],
        out_shape=[
            jax.ShapeDtypeStruct((B * N, 512), F32),
            jax.ShapeDtypeStruct((B * NB, 1, 512), F32),
            jax.ShapeDtypeStruct((B * NB, 1, 512), F32),
        ],
    )
    return f(g16, idx, dist, x2, l2W, l2b.reshape(1, 64), l1W,
             l1b.reshape(1, 64), c3WT, c3b.reshape(1, 128), mu, var,
             c3g.reshape(1, 128), c3be.reshape(1, 128), m1W1,
             m1b1.reshape(1, 512))


def _mlp2_body(R, z1_ref, mu_ref, var_ref, g_ref, be_ref, W2_ref, b2_ref,
               z2_ref, bsum_ref, bss_ref):
    den = jnp.sqrt(var_ref[...] + 1e-5)
    a1 = _leaky((z1_ref[...] - mu_ref[...]) / den * g_ref[...] + be_ref[...])
    z2 = _dot(a1.astype(BF), W2_ref[...].astype(BF)) + b2_ref[...]
    z2_ref[...] = z2
    bs = jnp.sum(z2, axis=0, keepdims=True)
    bsum_ref[0] = bs
    zc = z2 - bs / R
    bss_ref[0] = jnp.sum(zc * zc, axis=0, keepdims=True)


def _mlp2_pass(z1, mu1, var1, g1, be1, m1W2, m1b2, R):
    M = z1.shape[0]
    NB = M // R
    f = pl.pallas_call(
        functools.partial(_mlp2_body, R),
        grid=(NB,),
        in_specs=[
            pl.BlockSpec((R, 512), lambda i: (i, 0)),
            pl.BlockSpec((1, 512), lambda i: (0, 0)),
            pl.BlockSpec((1, 512), lambda i: (0, 0)),
            pl.BlockSpec((1, 512), lambda i: (0, 0)),
            pl.BlockSpec((1, 512), lambda i: (0, 0)),
            pl.BlockSpec((512, 512), lambda i: (0, 0)),
            pl.BlockSpec((1, 512), lambda i: (0, 0)),
        ],
        out_specs=[
            pl.BlockSpec((R, 512), lambda i: (i, 0)),
            pl.BlockSpec((1, 1, 512), lambda i: (i, 0, 0)),
            pl.BlockSpec((1, 1, 512), lambda i: (i, 0, 0)),
        ],
        out_shape=[
            jax.ShapeDtypeStruct((M, 512), F32),
            jax.ShapeDtypeStruct((NB, 1, 512), F32),
            jax.ShapeDtypeStruct((NB, 1, 512), F32),
        ],
    )
    return f(z1, mu1, var1, g1.reshape(1, 512), be1.reshape(1, 512), m1W2,
             m1b2.reshape(1, 512))


def _final_body(z2_ref, mu_ref, var_ref, g_ref, be_ref, o_ref):
    den = jnp.sqrt(var_ref[...] + 1e-5)
    o_ref[...] = _leaky(
        (z2_ref[...] - mu_ref[...]) / den * g_ref[...] + be_ref[...])


def _final_pass(z2, mu2, var2, g2, be2, R):
    M = z2.shape[0]
    NB = M // R
    f = pl.pallas_call(
        _final_body,
        grid=(NB,),
        in_specs=[
            pl.BlockSpec((R, 512), lambda i: (i, 0)),
            pl.BlockSpec((1, 512), lambda i: (0, 0)),
            pl.BlockSpec((1, 512), lambda i: (0, 0)),
            pl.BlockSpec((1, 512), lambda i: (0, 0)),
            pl.BlockSpec((1, 512), lambda i: (0, 0)),
        ],
        out_specs=[pl.BlockSpec((R, 512), lambda i: (i, 0))],
        out_shape=[jax.ShapeDtypeStruct((M, 512), F32)],
    )
    return f(z2, mu2, var2, g2.reshape(1, 512), be2.reshape(1, 512))[0]


def _combine_stats(bsum, bss, m, cnt):
    bsum = bsum.reshape(bsum.shape[0], -1)
    bss = bss.reshape(bss.shape[0], -1)
    mu = jnp.sum(bsum, axis=0, keepdims=True) / cnt
    bm = bsum / m
    var = (jnp.sum(bss, axis=0, keepdims=True)
           + m * jnp.sum((bm - mu) ** 2, axis=0, keepdims=True)) / cnt
    return mu, var


def kernel(x, W1, b1, W2, b2, l1W, l1b, l2W, l2b, c3W, c3b, c3g, c3be,
           m1W1, m1b1, m1g1, m1be1, m1W2, m1b2, m1g2, m1be2):
    B, N, _ = x.shape
    R = 128
    Rk = 256
    dist, idx, x1 = _knn_edge(x, W1, b1, Rk, emit_knn=True)
    x2, = _knn_edge(x1, W2, b2, Rk, emit_knn=False)

    c3WT = c3W.T
    g16, bsum, bss = _stats_pass(idx, dist, x2, l2W, l2b, l1W, l1b, c3WT,
                                 c3b, R)
    mu, var = _combine_stats(bsum, bss, KNN * R, B * N * KNN)

    z1, bsum1, bss1 = _score_pass(g16, idx, dist, x2, l2W, l2b, l1W, l1b,
                                  c3WT, c3b, mu, var, c3g, c3be, m1W1,
                                  m1b1, R)
    mu1, var1 = _combine_stats(bsum1, bss1, R, B * N)

    Rz = 512
    z2, bsum2, bss2 = _mlp2_pass(z1, mu1, var1, m1g1, m1be1, m1W2, m1b2, Rz)
    mu2, var2 = _combine_stats(bsum2, bss2, Rz, B * N)

    out = _final_pass(z2, mu2, var2, m1g2, m1be2, Rz)
    return out.reshape(B, N, 512)


# speedup vs baseline: 4.7945x; 1.0279x over previous
import functools

import jax
import jax.numpy as jnp
from jax.experimental import pallas as pl
from jax.experimental.pallas import tpu as pltpu

KNN = 33
NKEY = 10
BIG = 3e38
F32 = jnp.float32
BF = jnp.bfloat16


def _split3(x):
    hi = x.astype(BF)
    r1 = x - hi.astype(F32)
    mid = r1.astype(BF)
    lo = (r1 - mid.astype(F32)).astype(BF)
    return hi, mid, lo


def _leaky(x):
    return jnp.where(x >= 0, x, 0.2 * x)


def _dot(a, b):
    return jnp.dot(a, b, preferred_element_type=F32)


def _knn_edge_body(R, N, F, emit_knn, xr_ref, xt_ref, xc_ref, W_ref, b_ref,
                   *out_refs):
    if emit_knn:
        dist_ref, idx_ref, xo_ref = out_refs
    else:
        xo_ref, = out_refs
    xr = xr_ref[0]
    xt = xt_ref[0]
    xc = xc_ref[0]
    sqj = jnp.zeros((1, N), F32)
    for c in range(F):
        row = xt[c:c + 1, :]
        sqj = sqj + row * row
    sqi = jnp.sum(xr * xr, axis=1, keepdims=True)
    dot = _dot(xr.astype(BF), xt.astype(BF))
    d = (sqi + sqj) - 2.0 * dot
    hi, mid, lo = _split3(xc)
    tbl = jnp.concatenate([hi, mid, lo], axis=1)
    W16 = W_ref[...].astype(BF)
    brow = b_ref[...]
    iota = jax.lax.broadcasted_iota(jnp.int32, (R, N), 1)
    rowid = pl.program_id(1) * R + jax.lax.broadcasted_iota(
        jnp.int32, (R, 1), 0)
    acc = jnp.full((R, W_ref.shape[1]), -BIG, F32)
    dd = d
    dcols, icols = [], []
    for t in range(KNN):
        m = jnp.min(dd, axis=1, keepdims=True)
        a = jnp.min(jnp.where(dd == m, iota, N), axis=1, keepdims=True)
        oh = iota == a
        dd = jnp.where(oh, BIG, dd)
        if emit_knn:
            dcols.append(m)
            icols.append(a)
        oh16 = oh.astype(BF)
        parts = _dot(oh16, tbl)
        xj = (parts[:, :F] + parts[:, F:2 * F]) + parts[:, 2 * F:]
        cc = jnp.concatenate([xr, xj - xr], axis=1).astype(BF)
        msg = _dot(cc, W16) + brow
        msg = _leaky(msg)
        acc = jnp.maximum(acc, jnp.where(a != rowid, msg, -BIG))
    if emit_knn:
        dist_ref[0] = jnp.concatenate(dcols, axis=1)
        idx_ref[0] = jnp.concatenate(icols, axis=1)
    xo_ref[0] = acc


def _knn_edge(x, W, b, R, emit_knn):
    B, N, F = x.shape
    Fo = W.shape[1]
    xt = x.transpose(0, 2, 1)
    grid = (B, N // R)
    in_specs = [
        pl.BlockSpec((1, R, F), lambda bb, nb: (bb, nb, 0)),
        pl.BlockSpec((1, F, N), lambda bb, nb: (bb, 0, 0)),
        pl.BlockSpec((1, N, F), lambda bb, nb: (bb, 0, 0)),
        pl.BlockSpec((2 * F, Fo), lambda bb, nb: (0, 0)),
        pl.BlockSpec((1, Fo), lambda bb, nb: (0, 0)),
    ]
    if emit_knn:
        out_shape = [
            jax.ShapeDtypeStruct((B, N, KNN), F32),
            jax.ShapeDtypeStruct((B, N, KNN), jnp.int32),
            jax.ShapeDtypeStruct((B, N, Fo), F32),
        ]
        out_specs = [
            pl.BlockSpec((1, R, KNN), lambda bb, nb: (bb, nb, 0)),
            pl.BlockSpec((1, R, KNN), lambda bb, nb: (bb, nb, 0)),
            pl.BlockSpec((1, R, Fo), lambda bb, nb: (bb, nb, 0)),
        ]
    else:
        out_shape = [jax.ShapeDtypeStruct((B, N, Fo), F32)]
        out_specs = [pl.BlockSpec((1, R, Fo), lambda bb, nb: (bb, nb, 0))]
    f = pl.pallas_call(
        functools.partial(_knn_edge_body, R, N, F, emit_knn),
        grid=grid,
        in_specs=in_specs,
        out_specs=out_specs,
        out_shape=out_shape,
    )
    return f(x, xt, x, W, b.reshape(1, Fo))


def _hk(g16_k, ecol, l2W16, l2b, l1w, l1b, c3WT16, c3b):
    knnf = _dot(g16_k, l2W16) + l2b
    distf = ecol * l1w + l1b
    feat = jnp.concatenate([knnf, distf], axis=1)
    return _dot(feat.astype(BF), c3WT16) + c3b


def _stats_body(R, N, idx_ref, dist_ref, x2_ref, l2W_ref, l2b_ref, l1w_ref,
                l1b_ref, c3WT_ref, c3b_ref, g16_ref, bsum_ref, bss_ref,
                h_scr):
    x216 = x2_ref[0].astype(BF)
    l2W16 = l2W_ref[...].astype(BF)
    c3WT16 = c3WT_ref[...].astype(BF)
    l2b = l2b_ref[...]
    l1w = l1w_ref[...]
    l1b = l1b_ref[...]
    c3b = c3b_ref[...]
    iota = jax.lax.broadcasted_iota(jnp.int32, (R, N), 1)
    s = jnp.zeros((1, 128), F32)
    for k in range(KNN):
        oh = (iota == idx_ref[0][:, k:k + 1]).astype(BF)
        g_k = _dot(oh, x216)
        g16_ref[k] = g_k.astype(BF)
        ecol = jnp.exp(-dist_ref[0][:, k:k + 1])
        h = _hk(g_k.astype(BF), ecol, l2W16, l2b, l1w, l1b, c3WT16, c3b)
        h_scr[k * R:(k + 1) * R, :] = h
        s = s + jnp.sum(h, axis=0, keepdims=True)
    bsum_ref[0] = s
    bm = s / (KNN * R)
    ss = jnp.zeros((1, 128), F32)
    for k in range(KNN):
        hc = h_scr[k * R:(k + 1) * R, :] - bm
        ss = ss + jnp.sum(hc * hc, axis=0, keepdims=True)
    bss_ref[0] = ss


def _stats_pass(idx, dist, x2, l2W, l2b, l1W, l1b, c3WT, c3b, R):
    B, N, _ = x2.shape
    NB = N // R
    grid = (B, NB)
    f = pl.pallas_call(
        functools.partial(_stats_body, R, N),
        grid=grid,
        in_specs=[
            pl.BlockSpec((1, R, KNN), lambda bb, nb: (bb, nb, 0)),
            pl.BlockSpec((1, R, KNN), lambda bb, nb: (bb, nb, 0)),
            pl.BlockSpec((1, N, 48), lambda bb, nb: (bb, 0, 0)),
            pl.BlockSpec((48, 64), lambda bb, nb: (0, 0)),
            pl.BlockSpec((1, 64), lambda bb, nb: (0, 0)),
            pl.BlockSpec((1, 64), lambda bb, nb: (0, 0)),
            pl.BlockSpec((1, 64), lambda bb, nb: (0, 0)),
            pl.BlockSpec((128, 128), lambda bb, nb: (0, 0)),
            pl.BlockSpec((1, 128), lambda bb, nb: (0, 0)),
        ],
        out_specs=[
            pl.BlockSpec((KNN, R, 48),
                         lambda bb, nb, NB=NB: (0, bb * NB + nb, 0)),
            pl.BlockSpec((1, 1, 128), lambda bb, nb, NB=NB: (bb * NB + nb, 0, 0)),
            pl.BlockSpec((1, 1, 128), lambda bb, nb, NB=NB: (bb * NB + nb, 0, 0)),
        ],
        out_shape=[
            jax.ShapeDtypeStruct((KNN, B * N, 48), BF),
            jax.ShapeDtypeStruct((B * NB, 1, 128), F32),
            jax.ShapeDtypeStruct((B * NB, 1, 128), F32),
        ],
        scratch_shapes=[pltpu.VMEM((KNN * R, 128), F32)],
    )
    return f(idx, dist, x2, l2W, l2b.reshape(1, 64), l1W, l1b.reshape(1, 64),
             c3WT, c3b.reshape(1, 128))


def _score_body(R, N, g16_ref, idx_ref, dist_ref, x2_ref, l2W_ref, l2b_ref,
                l1w_ref, l1b_ref, c3WT_ref, c3b_ref, mu_ref, var_ref, g_ref,
                be_ref, W1_ref, b1_ref, z1_ref, bsum_ref, bss_ref):
    l2W16 = l2W_ref[...].astype(BF)
    c3WT16 = c3WT_ref[...].astype(BF)
    l2b = l2b_ref[...]
    l1w = l1w_ref[...]
    l1b = l1b_ref[...]
    c3b = c3b_ref[...]
    mu = mu_ref[...]
    den = jnp.sqrt(var_ref[...] + 1e-5)
    gg = g_ref[...]
    be = be_ref[...]
    scols = []
    for k in range(KNN):
        g_k = g16_ref[k]
        ecol = jnp.exp(-dist_ref[0][:, k:k + 1])
        h = _hk(g_k, ecol, l2W16, l2b, l1w, l1b, c3WT16, c3b)
        hh = _leaky((h - mu) / den * gg + be)
        scols.append(jnp.max(hh, axis=1, keepdims=True))
    score = jnp.concatenate(scols, axis=1)
    idxs = idx_ref[0]
    it = jax.lax.broadcasted_iota(jnp.int32, (R, KNN), 1)
    x216 = x2_ref[0].astype(BF)
    iota = jax.lax.broadcasted_iota(jnp.int32, (R, N), 1)
    W116 = W1_ref[...].astype(BF)
    z1 = jnp.zeros((R, 512), F32)
    s = score
    for t in range(NKEY):
        m = jnp.max(s, axis=1, keepdims=True)
        a = jnp.min(jnp.where(s == m, it, KNN), axis=1, keepdims=True)
        ohk = it == a
        s = jnp.where(ohk, -BIG, s)
        tp = jnp.sum(jnp.where(ohk, idxs, 0), axis=1, keepdims=True)
        oh = (iota == tp).astype(BF)
        keyf = _dot(oh, x216).astype(BF)
        z1 = z1 + _dot(keyf, W116[t * 48:(t + 1) * 48, :])
    z1 = z1 + b1_ref[...]
    z1_ref[...] = z1
    bs = jnp.sum(z1, axis=0, keepdims=True)
    bsum_ref[0] = bs
    zc = z1 - bs / R
    bss_ref[0] = jnp.sum(zc * zc, axis=0, keepdims=True)


def _score_pass(g16, idx, dist, x2, l2W, l2b, l1W, l1b, c3WT, c3b, mu, var,
                c3g, c3be, m1W1, m1b1, R):
    B, N, _ = x2.shape
    NB = N // R
    grid = (B, NB)
    f = pl.pallas_call(
        functools.partial(_score_body, R, N),
        grid=grid,
        in_specs=[
            pl.BlockSpec((KNN, R, 48), lambda bb, nb, NB=NB: (0, bb * NB + nb, 0)),
            pl.BlockSpec((1, R, KNN), lambda bb, nb: (bb, nb, 0)),
            pl.BlockSpec((1, R, KNN), lambda bb, nb: (bb, nb, 0)),
            pl.BlockSpec((1, N, 48), lambda bb, nb: (bb, 0, 0)),
            pl.BlockSpec((48, 64), lambda bb, nb: (0, 0)),
            pl.BlockSpec((1, 64), lambda bb, nb: (0, 0)),
            pl.BlockSpec((1, 64), lambda bb, nb: (0, 0)),
            pl.BlockSpec((1, 64), lambda bb, nb: (0, 0)),
            pl.BlockSpec((128, 128), lambda bb, nb: (0, 0)),
            pl.BlockSpec((1, 128), lambda bb, nb: (0, 0)),
            pl.BlockSpec((1, 128), lambda bb, nb: (0, 0)),
            pl.BlockSpec((1, 128), lambda bb, nb: (0, 0)),
            pl.BlockSpec((1, 128), lambda bb, nb: (0, 0)),
            pl.BlockSpec((1, 128), lambda bb, nb: (0, 0)),
            pl.BlockSpec((480, 512), lambda bb, nb: (0, 0)),
            pl.BlockSpec((1, 512), lambda bb, nb: (0, 0)),
        ],
        out_specs=[
            pl.BlockSpec((R, 512), lambda bb, nb, NB=NB: (bb * NB + nb, 0)),
            pl.BlockSpec((1, 1, 512), lambda bb, nb, NB=NB: (bb * NB + nb, 0, 0)),
            pl.BlockSpec((1, 1, 512), lambda bb, nb, NB=NB: (bb * NB + nb, 0, 0)),
        ],
        out_shape=[
            jax.ShapeDtypeStruct((B * N, 512), F32),
            jax.ShapeDtypeStruct((B * NB, 1, 512), F32),
            jax.ShapeDtypeStruct((B * NB, 1, 512), F32),
        ],
    )
    return f(g16, idx, dist, x2, l2W, l2b.reshape(1, 64), l1W,
             l1b.reshape(1, 64), c3WT, c3b.reshape(1, 128), mu, var,
             c3g.reshape(1, 128), c3be.reshape(1, 128), m1W1,
             m1b1.reshape(1, 512))


def _mlp2_body(R, z1_ref, mu_ref, var_ref, g_ref, be_ref, W2_ref, b2_ref,
               z2_ref, bsum_ref, bss_ref):
    den = jnp.sqrt(var_ref[...] + 1e-5)
    a1 = _leaky((z1_ref[...] - mu_ref[...]) / den * g_ref[...] + be_ref[...])
    z2 = _dot(a1.astype(BF), W2_ref[...].astype(BF)) + b2_ref[...]
    z2_ref[...] = z2
    bs = jnp.sum(z2, axis=0, keepdims=True)
    bsum_ref[0] = bs
    zc = z2 - bs / R
    bss_ref[0] = jnp.sum(zc * zc, axis=0, keepdims=True)


def _mlp2_pass(z1, mu1, var1, g1, be1, m1W2, m1b2, R):
    M = z1.shape[0]
    NB = M // R
    f = pl.pallas_call(
        functools.partial(_mlp2_body, R),
        grid=(NB,),
        in_specs=[
            pl.BlockSpec((R, 512), lambda i: (i, 0)),
            pl.BlockSpec((1, 512), lambda i: (0, 0)),
            pl.BlockSpec((1, 512), lambda i: (0, 0)),
            pl.BlockSpec((1, 512), lambda i: (0, 0)),
            pl.BlockSpec((1, 512), lambda i: (0, 0)),
            pl.BlockSpec((512, 512), lambda i: (0, 0)),
            pl.BlockSpec((1, 512), lambda i: (0, 0)),
        ],
        out_specs=[
            pl.BlockSpec((R, 512), lambda i: (i, 0)),
            pl.BlockSpec((1, 1, 512), lambda i: (i, 0, 0)),
            pl.BlockSpec((1, 1, 512), lambda i: (i, 0, 0)),
        ],
        out_shape=[
            jax.ShapeDtypeStruct((M, 512), F32),
            jax.ShapeDtypeStruct((NB, 1, 512), F32),
            jax.ShapeDtypeStruct((NB, 1, 512), F32),
        ],
    )
    return f(z1, mu1, var1, g1.reshape(1, 512), be1.reshape(1, 512), m1W2,
             m1b2.reshape(1, 512))


def _final_body(z2_ref, mu_ref, var_ref, g_ref, be_ref, o_ref):
    den = jnp.sqrt(var_ref[...] + 1e-5)
    o_ref[...] = _leaky(
        (z2_ref[...] - mu_ref[...]) / den * g_ref[...] + be_ref[...])


def _final_pass(z2, mu2, var2, g2, be2, R):
    M = z2.shape[0]
    NB = M // R
    f = pl.pallas_call(
        _final_body,
        grid=(NB,),
        in_specs=[
            pl.BlockSpec((R, 512), lambda i: (i, 0)),
            pl.BlockSpec((1, 512), lambda i: (0, 0)),
            pl.BlockSpec((1, 512), lambda i: (0, 0)),
            pl.BlockSpec((1, 512), lambda i: (0, 0)),
            pl.BlockSpec((1, 512), lambda i: (0, 0)),
        ],
        out_specs=[pl.BlockSpec((R, 512), lambda i: (i, 0))],
        out_shape=[jax.ShapeDtypeStruct((M, 512), F32)],
    )
    return f(z2, mu2, var2, g2.reshape(1, 512), be2.reshape(1, 512))[0]


def _combine_stats(bsum, bss, m, cnt):
    bsum = bsum.reshape(bsum.shape[0], -1)
    bss = bss.reshape(bss.shape[0], -1)
    mu = jnp.sum(bsum, axis=0, keepdims=True) / cnt
    bm = bsum / m
    var = (jnp.sum(bss, axis=0, keepdims=True)
           + m * jnp.sum((bm - mu) ** 2, axis=0, keepdims=True)) / cnt
    return mu, var


def kernel(x, W1, b1, W2, b2, l1W, l1b, l2W, l2b, c3W, c3b, c3g, c3be,
           m1W1, m1b1, m1g1, m1be1, m1W2, m1b2, m1g2, m1be2):
    B, N, _ = x.shape
    R = 256
    Rk = 512
    dist, idx, x1 = _knn_edge(x, W1, b1, Rk, emit_knn=True)
    x2, = _knn_edge(x1, W2, b2, Rk, emit_knn=False)

    c3WT = c3W.T
    g16, bsum, bss = _stats_pass(idx, dist, x2, l2W, l2b, l1W, l1b, c3WT,
                                 c3b, R)
    mu, var = _combine_stats(bsum, bss, KNN * R, B * N * KNN)

    z1, bsum1, bss1 = _score_pass(g16, idx, dist, x2, l2W, l2b, l1W, l1b,
                                  c3WT, c3b, mu, var, c3g, c3be, m1W1,
                                  m1b1, R)
    mu1, var1 = _combine_stats(bsum1, bss1, R, B * N)

    Rz = 512
    z2, bsum2, bss2 = _mlp2_pass(z1, mu1, var1, m1g1, m1be1, m1W2, m1b2, Rz)
    mu2, var2 = _combine_stats(bsum2, bss2, Rz, B * N)

    out = _final_pass(z2, mu2, var2, m1g2, m1be2, Rz)
    return out.reshape(B, N, 512)


# speedup vs baseline: 4.8993x; 1.0219x over previous
import functools

import jax
import jax.numpy as jnp
from jax.experimental import pallas as pl
from jax.experimental.pallas import tpu as pltpu

KNN = 33
NKEY = 10
BIG = 3e38
F32 = jnp.float32
BF = jnp.bfloat16


def _split3(x):
    hi = x.astype(BF)
    r1 = x - hi.astype(F32)
    mid = r1.astype(BF)
    lo = (r1 - mid.astype(F32)).astype(BF)
    return hi, mid, lo


def _leaky(x):
    return jnp.where(x >= 0, x, 0.2 * x)


def _dot(a, b):
    return jnp.dot(a, b, preferred_element_type=F32)


def _knn_edge_body(R, N, F, emit_knn, xr_ref, xt_ref, xc_ref, W_ref, b_ref,
                   *out_refs):
    if emit_knn:
        dist_ref, idx_ref, xo_ref = out_refs
    else:
        xo_ref, = out_refs
    xr = xr_ref[0]
    xt = xt_ref[0]
    xc = xc_ref[0]
    sqj = jnp.zeros((1, N), F32)
    for c in range(F):
        row = xt[c:c + 1, :]
        sqj = sqj + row * row
    sqi = jnp.sum(xr * xr, axis=1, keepdims=True)
    dot = _dot(xr.astype(BF), xt.astype(BF))
    d = (sqi + sqj) - 2.0 * dot
    hi, mid, lo = _split3(xc)
    tbl = jnp.concatenate([hi, mid, lo], axis=1)
    W16 = W_ref[...].astype(BF)
    brow = b_ref[...]
    iota = jax.lax.broadcasted_iota(jnp.int32, (R, N), 1)
    rowid = pl.program_id(1) * R + jax.lax.broadcasted_iota(
        jnp.int32, (R, 1), 0)
    acc = jnp.full((R, W_ref.shape[1]), -BIG, F32)
    dd = d
    dcols, icols = [], []
    for t in range(KNN):
        m = jnp.min(dd, axis=1, keepdims=True)
        a = jnp.min(jnp.where(dd == m, iota, N), axis=1, keepdims=True)
        oh = iota == a
        dd = jnp.where(oh, BIG, dd)
        if emit_knn:
            dcols.append(m)
            icols.append(a)
        oh16 = oh.astype(BF)
        parts = _dot(oh16, tbl)
        xj = (parts[:, :F] + parts[:, F:2 * F]) + parts[:, 2 * F:]
        cc = jnp.concatenate([xr, xj - xr], axis=1).astype(BF)
        msg = _dot(cc, W16) + brow
        msg = _leaky(msg)
        acc = jnp.maximum(acc, jnp.where(a != rowid, msg, -BIG))
    if emit_knn:
        dist_ref[0] = jnp.concatenate(dcols, axis=1)
        idx_ref[0] = jnp.concatenate(icols, axis=1)
    xo_ref[0] = acc


def _knn_edge(x, W, b, R, emit_knn):
    B, N, F = x.shape
    Fo = W.shape[1]
    xt = x.transpose(0, 2, 1)
    grid = (B, N // R)
    in_specs = [
        pl.BlockSpec((1, R, F), lambda bb, nb: (bb, nb, 0)),
        pl.BlockSpec((1, F, N), lambda bb, nb: (bb, 0, 0)),
        pl.BlockSpec((1, N, F), lambda bb, nb: (bb, 0, 0)),
        pl.BlockSpec((2 * F, Fo), lambda bb, nb: (0, 0)),
        pl.BlockSpec((1, Fo), lambda bb, nb: (0, 0)),
    ]
    if emit_knn:
        out_shape = [
            jax.ShapeDtypeStruct((B, N, KNN), F32),
            jax.ShapeDtypeStruct((B, N, KNN), jnp.int32),
            jax.ShapeDtypeStruct((B, N, Fo), F32),
        ]
        out_specs = [
            pl.BlockSpec((1, R, KNN), lambda bb, nb: (bb, nb, 0)),
            pl.BlockSpec((1, R, KNN), lambda bb, nb: (bb, nb, 0)),
            pl.BlockSpec((1, R, Fo), lambda bb, nb: (bb, nb, 0)),
        ]
    else:
        out_shape = [jax.ShapeDtypeStruct((B, N, Fo), F32)]
        out_specs = [pl.BlockSpec((1, R, Fo), lambda bb, nb: (bb, nb, 0))]
    f = pl.pallas_call(
        functools.partial(_knn_edge_body, R, N, F, emit_knn),
        grid=grid,
        in_specs=in_specs,
        out_specs=out_specs,
        out_shape=out_shape,
    )
    return f(x, xt, x, W, b.reshape(1, Fo))


def _hk(g16_k, ecol, l2W16, l2b, l1w, l1b, c3WT16, c3b):
    knnf = _dot(g16_k, l2W16) + l2b
    distf = ecol * l1w + l1b
    return (_dot(knnf.astype(BF), c3WT16[:64, :])
            + _dot(distf.astype(BF), c3WT16[64:, :])) + c3b


def _stats_body(R, N, idx_ref, dist_ref, x2_ref, l2W_ref, l2b_ref, l1w_ref,
                l1b_ref, c3WT_ref, c3b_ref, g16_ref, bsum_ref, bss_ref,
                h_scr):
    x216 = x2_ref[0].astype(BF)
    l2W16 = l2W_ref[...].astype(BF)
    c3WT16 = c3WT_ref[...].astype(BF)
    l2b = l2b_ref[...]
    l1w = l1w_ref[...]
    l1b = l1b_ref[...]
    c3b = c3b_ref[...]
    iota = jax.lax.broadcasted_iota(jnp.int32, (R, N), 1)
    s = jnp.zeros((1, 128), F32)
    for k in range(KNN):
        oh = (iota == idx_ref[0][:, k:k + 1]).astype(BF)
        g_k = _dot(oh, x216)
        g16_ref[k] = g_k.astype(BF)
        ecol = jnp.exp(-dist_ref[0][:, k:k + 1])
        h = _hk(g_k.astype(BF), ecol, l2W16, l2b, l1w, l1b, c3WT16, c3b)
        h_scr[k * R:(k + 1) * R, :] = h
        s = s + jnp.sum(h, axis=0, keepdims=True)
    bsum_ref[0] = s
    bm = s / (KNN * R)
    ss = jnp.zeros((1, 128), F32)
    for k in range(KNN):
        hc = h_scr[k * R:(k + 1) * R, :] - bm
        ss = ss + jnp.sum(hc * hc, axis=0, keepdims=True)
    bss_ref[0] = ss


def _stats_pass(idx, dist, x2, l2W, l2b, l1W, l1b, c3WT, c3b, R):
    B, N, _ = x2.shape
    NB = N // R
    grid = (B, NB)
    f = pl.pallas_call(
        functools.partial(_stats_body, R, N),
        grid=grid,
        in_specs=[
            pl.BlockSpec((1, R, KNN), lambda bb, nb: (bb, nb, 0)),
            pl.BlockSpec((1, R, KNN), lambda bb, nb: (bb, nb, 0)),
            pl.BlockSpec((1, N, 48), lambda bb, nb: (bb, 0, 0)),
            pl.BlockSpec((48, 64), lambda bb, nb: (0, 0)),
            pl.BlockSpec((1, 64), lambda bb, nb: (0, 0)),
            pl.BlockSpec((1, 64), lambda bb, nb: (0, 0)),
            pl.BlockSpec((1, 64), lambda bb, nb: (0, 0)),
            pl.BlockSpec((128, 128), lambda bb, nb: (0, 0)),
            pl.BlockSpec((1, 128), lambda bb, nb: (0, 0)),
        ],
        out_specs=[
            pl.BlockSpec((KNN, R, 48),
                         lambda bb, nb, NB=NB: (0, bb * NB + nb, 0)),
            pl.BlockSpec((1, 1, 128), lambda bb, nb, NB=NB: (bb * NB + nb, 0, 0)),
            pl.BlockSpec((1, 1, 128), lambda bb, nb, NB=NB: (bb * NB + nb, 0, 0)),
        ],
        out_shape=[
            jax.ShapeDtypeStruct((KNN, B * N, 48), BF),
            jax.ShapeDtypeStruct((B * NB, 1, 128), F32),
            jax.ShapeDtypeStruct((B * NB, 1, 128), F32),
        ],
        scratch_shapes=[pltpu.VMEM((KNN * R, 128), F32)],
    )
    return f(idx, dist, x2, l2W, l2b.reshape(1, 64), l1W, l1b.reshape(1, 64),
             c3WT, c3b.reshape(1, 128))


def _score_body(R, N, g16_ref, idx_ref, dist_ref, x2_ref, l2W_ref, l2b_ref,
                l1w_ref, l1b_ref, c3WT_ref, c3b_ref, mu_ref, var_ref, g_ref,
                be_ref, W1_ref, b1_ref, z1_ref, bsum_ref, bss_ref):
    l2W16 = l2W_ref[...].astype(BF)
    c3WT16 = c3WT_ref[...].astype(BF)
    l2b = l2b_ref[...]
    l1w = l1w_ref[...]
    l1b = l1b_ref[...]
    c3b = c3b_ref[...]
    mu = mu_ref[...]
    den = jnp.sqrt(var_ref[...] + 1e-5)
    gg = g_ref[...]
    be = be_ref[...]
    scols = []
    for k in range(KNN):
        g_k = g16_ref[k]
        ecol = jnp.exp(-dist_ref[0][:, k:k + 1])
        h = _hk(g_k, ecol, l2W16, l2b, l1w, l1b, c3WT16, c3b)
        hh = _leaky((h - mu) / den * gg + be)
        scols.append(jnp.max(hh, axis=1, keepdims=True))
    score = jnp.concatenate(scols, axis=1)
    idxs = idx_ref[0]
    it = jax.lax.broadcasted_iota(jnp.int32, (R, KNN), 1)
    x216 = x2_ref[0].astype(BF)
    iota = jax.lax.broadcasted_iota(jnp.int32, (R, N), 1)
    W116 = W1_ref[...].astype(BF)
    z1 = jnp.zeros((R, 512), F32)
    s = score
    for t in range(NKEY):
        m = jnp.max(s, axis=1, keepdims=True)
        a = jnp.min(jnp.where(s == m, it, KNN), axis=1, keepdims=True)
        ohk = it == a
        s = jnp.where(ohk, -BIG, s)
        tp = jnp.sum(jnp.where(ohk, idxs, 0), axis=1, keepdims=True)
        oh = (iota == tp).astype(BF)
        keyf = _dot(oh, x216).astype(BF)
        z1 = z1 + _dot(keyf, W116[t * 48:(t + 1) * 48, :])
    z1 = z1 + b1_ref[...]
    z1_ref[...] = z1
    bs = jnp.sum(z1, axis=0, keepdims=True)
    bsum_ref[0] = bs
    zc = z1 - bs / R
    bss_ref[0] = jnp.sum(zc * zc, axis=0, keepdims=True)


def _score_pass(g16, idx, dist, x2, l2W, l2b, l1W, l1b, c3WT, c3b, mu, var,
                c3g, c3be, m1W1, m1b1, R):
    B, N, _ = x2.shape
    NB = N // R
    grid = (B, NB)
    f = pl.pallas_call(
        functools.partial(_score_body, R, N),
        grid=grid,
        in_specs=[
            pl.BlockSpec((KNN, R, 48), lambda bb, nb, NB=NB: (0, bb * NB + nb, 0)),
            pl.BlockSpec((1, R, KNN), lambda bb, nb: (bb, nb, 0)),
            pl.BlockSpec((1, R, KNN), lambda bb, nb: (bb, nb, 0)),
            pl.BlockSpec((1, N, 48), lambda bb, nb: (bb, 0, 0)),
            pl.BlockSpec((48, 64), lambda bb, nb: (0, 0)),
            pl.BlockSpec((1, 64), lambda bb, nb: (0, 0)),
            pl.BlockSpec((1, 64), lambda bb, nb: (0, 0)),
            pl.BlockSpec((1, 64), lambda bb, nb: (0, 0)),
            pl.BlockSpec((128, 128), lambda bb, nb: (0, 0)),
            pl.BlockSpec((1, 128), lambda bb, nb: (0, 0)),
            pl.BlockSpec((1, 128), lambda bb, nb: (0, 0)),
            pl.BlockSpec((1, 128), lambda bb, nb: (0, 0)),
            pl.BlockSpec((1, 128), lambda bb, nb: (0, 0)),
            pl.BlockSpec((1, 128), lambda bb, nb: (0, 0)),
            pl.BlockSpec((480, 512), lambda bb, nb: (0, 0)),
            pl.BlockSpec((1, 512), lambda bb, nb: (0, 0)),
        ],
        out_specs=[
            pl.BlockSpec((R, 512), lambda bb, nb, NB=NB: (bb * NB + nb, 0)),
            pl.BlockSpec((1, 1, 512), lambda bb, nb, NB=NB: (bb * NB + nb, 0, 0)),
            pl.BlockSpec((1, 1, 512), lambda bb, nb, NB=NB: (bb * NB + nb, 0, 0)),
        ],
        out_shape=[
            jax.ShapeDtypeStruct((B * N, 512), F32),
            jax.ShapeDtypeStruct((B * NB, 1, 512), F32),
            jax.ShapeDtypeStruct((B * NB, 1, 512), F32),
        ],
    )
    return f(g16, idx, dist, x2, l2W, l2b.reshape(1, 64), l1W,
             l1b.reshape(1, 64), c3WT, c3b.reshape(1, 128), mu, var,
             c3g.reshape(1, 128), c3be.reshape(1, 128), m1W1,
             m1b1.reshape(1, 512))


def _mlp2_body(R, z1_ref, mu_ref, var_ref, g_ref, be_ref, W2_ref, b2_ref,
               z2_ref, bsum_ref, bss_ref):
    den = jnp.sqrt(var_ref[...] + 1e-5)
    a1 = _leaky((z1_ref[...] - mu_ref[...]) / den * g_ref[...] + be_ref[...])
    z2 = _dot(a1.astype(BF), W2_ref[...].astype(BF)) + b2_ref[...]
    z2_ref[...] = z2
    bs = jnp.sum(z2, axis=0, keepdims=True)
    bsum_ref[0] = bs
    zc = z2 - bs / R
    bss_ref[0] = jnp.sum(zc * zc, axis=0, keepdims=True)


def _mlp2_pass(z1, mu1, var1, g1, be1, m1W2, m1b2, R):
    M = z1.shape[0]
    NB = M // R
    f = pl.pallas_call(
        functools.partial(_mlp2_body, R),
        grid=(NB,),
        in_specs=[
            pl.BlockSpec((R, 512), lambda i: (i, 0)),
            pl.BlockSpec((1, 512), lambda i: (0, 0)),
            pl.BlockSpec((1, 512), lambda i: (0, 0)),
            pl.BlockSpec((1, 512), lambda i: (0, 0)),
            pl.BlockSpec((1, 512), lambda i: (0, 0)),
            pl.BlockSpec((512, 512), lambda i: (0, 0)),
            pl.BlockSpec((1, 512), lambda i: (0, 0)),
        ],
        out_specs=[
            pl.BlockSpec((R, 512), lambda i: (i, 0)),
            pl.BlockSpec((1, 1, 512), lambda i: (i, 0, 0)),
            pl.BlockSpec((1, 1, 512), lambda i: (i, 0, 0)),
        ],
        out_shape=[
            jax.ShapeDtypeStruct((M, 512), F32),
            jax.ShapeDtypeStruct((NB, 1, 512), F32),
            jax.ShapeDtypeStruct((NB, 1, 512), F32),
        ],
    )
    return f(z1, mu1, var1, g1.reshape(1, 512), be1.reshape(1, 512), m1W2,
             m1b2.reshape(1, 512))


def _final_body(z2_ref, mu_ref, var_ref, g_ref, be_ref, o_ref):
    den = jnp.sqrt(var_ref[...] + 1e-5)
    o_ref[...] = _leaky(
        (z2_ref[...] - mu_ref[...]) / den * g_ref[...] + be_ref[...])


def _final_pass(z2, mu2, var2, g2, be2, R):
    M = z2.shape[0]
    NB = M // R
    f = pl.pallas_call(
        _final_body,
        grid=(NB,),
        in_specs=[
            pl.BlockSpec((R, 512), lambda i: (i, 0)),
            pl.BlockSpec((1, 512), lambda i: (0, 0)),
            pl.BlockSpec((1, 512), lambda i: (0, 0)),
            pl.BlockSpec((1, 512), lambda i: (0, 0)),
            pl.BlockSpec((1, 512), lambda i: (0, 0)),
        ],
        out_specs=[pl.BlockSpec((R, 512), lambda i: (i, 0))],
        out_shape=[jax.ShapeDtypeStruct((M, 512), F32)],
    )
    return f(z2, mu2, var2, g2.reshape(1, 512), be2.reshape(1, 512))[0]


def _combine_stats(bsum, bss, m, cnt):
    bsum = bsum.reshape(bsum.shape[0], -1)
    bss = bss.reshape(bss.shape[0], -1)
    mu = jnp.sum(bsum, axis=0, keepdims=True) / cnt
    bm = bsum / m
    var = (jnp.sum(bss, axis=0, keepdims=True)
           + m * jnp.sum((bm - mu) ** 2, axis=0, keepdims=True)) / cnt
    return mu, var


def kernel(x, W1, b1, W2, b2, l1W, l1b, l2W, l2b, c3W, c3b, c3g, c3be,
           m1W1, m1b1, m1g1, m1be1, m1W2, m1b2, m1g2, m1be2):
    B, N, _ = x.shape
    R = 256
    Rk = 512
    dist, idx, x1 = _knn_edge(x, W1, b1, Rk, emit_knn=True)
    x2, = _knn_edge(x1, W2, b2, Rk, emit_knn=False)

    c3WT = c3W.T
    g16, bsum, bss = _stats_pass(idx, dist, x2, l2W, l2b, l1W, l1b, c3WT,
                                 c3b, R)
    mu, var = _combine_stats(bsum, bss, KNN * R, B * N * KNN)

    z1, bsum1, bss1 = _score_pass(g16, idx, dist, x2, l2W, l2b, l1W, l1b,
                                  c3WT, c3b, mu, var, c3g, c3be, m1W1,
                                  m1b1, R)
    mu1, var1 = _combine_stats(bsum1, bss1, R, B * N)

    Rz = 512
    z2, bsum2, bss2 = _mlp2_pass(z1, mu1, var1, m1g1, m1be1, m1W2, m1b2, Rz)
    mu2, var2 = _combine_stats(bsum2, bss2, Rz, B * N)

    out = _final_pass(z2, mu2, var2, m1g2, m1be2, Rz)
    return out.reshape(B, N, 512)


# speedup vs baseline: 5.1735x; 1.0560x over previous
import functools

import jax
import jax.numpy as jnp
from jax.experimental import pallas as pl
from jax.experimental.pallas import tpu as pltpu

KNN = 33
NKEY = 10
BIG = 3e38
F32 = jnp.float32
BF = jnp.bfloat16


def _split3(x):
    hi = x.astype(BF)
    r1 = x - hi.astype(F32)
    mid = r1.astype(BF)
    lo = (r1 - mid.astype(F32)).astype(BF)
    return hi, mid, lo


def _leaky(x):
    return jnp.where(x >= 0, x, 0.2 * x)


def _dot(a, b):
    return jnp.dot(a, b, preferred_element_type=F32)


def _knn_edge_body(R, N, F, emit_knn, xr_ref, xt_ref, xc_ref, W_ref, b_ref,
                   *out_refs):
    if emit_knn:
        dist_ref, idx_ref, xo_ref = out_refs
    else:
        xo_ref, = out_refs
    xr = xr_ref[0]
    xt = xt_ref[0]
    xc = xc_ref[0]
    sqj = jnp.zeros((1, N), F32)
    for c in range(F):
        row = xt[c:c + 1, :]
        sqj = sqj + row * row
    sqi = jnp.sum(xr * xr, axis=1, keepdims=True)
    dot = _dot(xr.astype(BF), xt.astype(BF))
    d = (sqi + sqj) - 2.0 * dot
    hi, mid, lo = _split3(xc)
    tbl = jnp.concatenate([hi, mid, lo], axis=1)
    W16 = W_ref[...].astype(BF)
    brow = b_ref[...]
    iota = jax.lax.broadcasted_iota(jnp.int32, (R, N), 1)
    rowid = pl.program_id(1) * R + jax.lax.broadcasted_iota(
        jnp.int32, (R, 1), 0)
    acc = jnp.full((R, W_ref.shape[1]), -BIG, F32)
    dd = d
    dcols, icols = [], []
    for t in range(KNN):
        m = jnp.min(dd, axis=1, keepdims=True)
        a = jnp.min(jnp.where(dd == m, iota, N), axis=1, keepdims=True)
        oh = iota == a
        dd = jnp.where(oh, BIG, dd)
        if emit_knn:
            dcols.append(m)
            icols.append(a)
        oh16 = oh.astype(BF)
        parts = _dot(oh16, tbl)
        xj = (parts[:, :F] + parts[:, F:2 * F]) + parts[:, 2 * F:]
        cc = jnp.concatenate([xr, xj - xr], axis=1).astype(BF)
        msg = _dot(cc, W16) + brow
        msg = _leaky(msg)
        acc = jnp.maximum(acc, jnp.where(a != rowid, msg, -BIG))
    if emit_knn:
        dist_ref[0] = jnp.concatenate(dcols, axis=1)
        idx_ref[0] = jnp.concatenate(icols, axis=1)
    xo_ref[0] = acc


def _knn_edge(x, W, b, R, emit_knn):
    B, N, F = x.shape
    Fo = W.shape[1]
    xt = x.transpose(0, 2, 1)
    grid = (B, N // R)
    in_specs = [
        pl.BlockSpec((1, R, F), lambda bb, nb: (bb, nb, 0)),
        pl.BlockSpec((1, F, N), lambda bb, nb: (bb, 0, 0)),
        pl.BlockSpec((1, N, F), lambda bb, nb: (bb, 0, 0)),
        pl.BlockSpec((2 * F, Fo), lambda bb, nb: (0, 0)),
        pl.BlockSpec((1, Fo), lambda bb, nb: (0, 0)),
    ]
    if emit_knn:
        out_shape = [
            jax.ShapeDtypeStruct((B, N, KNN), F32),
            jax.ShapeDtypeStruct((B, N, KNN), jnp.int32),
            jax.ShapeDtypeStruct((B, N, Fo), F32),
        ]
        out_specs = [
            pl.BlockSpec((1, R, KNN), lambda bb, nb: (bb, nb, 0)),
            pl.BlockSpec((1, R, KNN), lambda bb, nb: (bb, nb, 0)),
            pl.BlockSpec((1, R, Fo), lambda bb, nb: (bb, nb, 0)),
        ]
    else:
        out_shape = [jax.ShapeDtypeStruct((B, N, Fo), F32)]
        out_specs = [pl.BlockSpec((1, R, Fo), lambda bb, nb: (bb, nb, 0))]
    f = pl.pallas_call(
        functools.partial(_knn_edge_body, R, N, F, emit_knn),
        grid=grid,
        in_specs=in_specs,
        out_specs=out_specs,
        out_shape=out_shape,
    )
    return f(x, xt, x, W, b.reshape(1, Fo))


def _hk(g16_k, ecol, l2W16, l2b, l1w, l1b, c3WT16, c3b):
    knnf = _dot(g16_k, l2W16) + l2b
    distf = ecol * l1w + l1b
    return (_dot(knnf.astype(BF), c3WT16[:64, :])
            + _dot(distf.astype(BF), c3WT16[64:, :])) + c3b


def _stats_body(R, N, idx_ref, dist_ref, x2_ref, l2W_ref, l2b_ref, l1w_ref,
                l1b_ref, c3WT_ref, c3b_ref, g16_ref, bsum_ref, bss_ref,
                h_scr):
    x216 = x2_ref[0].astype(BF)
    l2W16 = l2W_ref[...].astype(BF)
    c3WT16 = c3WT_ref[...].astype(BF)
    l2b = l2b_ref[...]
    l1w = l1w_ref[...]
    l1b = l1b_ref[...]
    c3b = c3b_ref[...]
    iota = jax.lax.broadcasted_iota(jnp.int32, (R, N), 1)
    s = jnp.zeros((1, 128), F32)
    for k in range(KNN):
        oh = (iota == idx_ref[0][:, k:k + 1]).astype(BF)
        g_k = _dot(oh, x216)
        g16_ref[k] = g_k.astype(BF)
        ecol = jnp.exp(-dist_ref[0][:, k:k + 1])
        h = _hk(g_k.astype(BF), ecol, l2W16, l2b, l1w, l1b, c3WT16, c3b)
        h_scr[k * R:(k + 1) * R, :] = h
        s = s + jnp.sum(h, axis=0, keepdims=True)
    bsum_ref[0] = s
    bm = s / (KNN * R)
    ss = jnp.zeros((1, 128), F32)
    for k in range(KNN):
        hc = h_scr[k * R:(k + 1) * R, :] - bm
        ss = ss + jnp.sum(hc * hc, axis=0, keepdims=True)
    bss_ref[0] = ss


def _stats_pass(idx, dist, x2, l2W, l2b, l1W, l1b, c3WT, c3b, R):
    B, N, _ = x2.shape
    NB = N // R
    grid = (B, NB)
    f = pl.pallas_call(
        functools.partial(_stats_body, R, N),
        grid=grid,
        in_specs=[
            pl.BlockSpec((1, R, KNN), lambda bb, nb: (bb, nb, 0)),
            pl.BlockSpec((1, R, KNN), lambda bb, nb: (bb, nb, 0)),
            pl.BlockSpec((1, N, 48), lambda bb, nb: (bb, 0, 0)),
            pl.BlockSpec((48, 64), lambda bb, nb: (0, 0)),
            pl.BlockSpec((1, 64), lambda bb, nb: (0, 0)),
            pl.BlockSpec((1, 64), lambda bb, nb: (0, 0)),
            pl.BlockSpec((1, 64), lambda bb, nb: (0, 0)),
            pl.BlockSpec((128, 128), lambda bb, nb: (0, 0)),
            pl.BlockSpec((1, 128), lambda bb, nb: (0, 0)),
        ],
        out_specs=[
            pl.BlockSpec((KNN, R, 48),
                         lambda bb, nb, NB=NB: (0, bb * NB + nb, 0)),
            pl.BlockSpec((1, 1, 128), lambda bb, nb, NB=NB: (bb * NB + nb, 0, 0)),
            pl.BlockSpec((1, 1, 128), lambda bb, nb, NB=NB: (bb * NB + nb, 0, 0)),
        ],
        out_shape=[
            jax.ShapeDtypeStruct((KNN, B * N, 48), BF),
            jax.ShapeDtypeStruct((B * NB, 1, 128), F32),
            jax.ShapeDtypeStruct((B * NB, 1, 128), F32),
        ],
        scratch_shapes=[pltpu.VMEM((KNN * R, 128), F32)],
    )
    return f(idx, dist, x2, l2W, l2b.reshape(1, 64), l1W, l1b.reshape(1, 64),
             c3WT, c3b.reshape(1, 128))


def _score_body(R, N, g16_ref, idx_ref, dist_ref, x2_ref, l2W_ref, l2b_ref,
                l1w_ref, l1b_ref, c3WT_ref, c3b_ref, mu_ref, var_ref, g_ref,
                be_ref, W1_ref, b1_ref, z1_ref, bsum_ref, bss_ref):
    l2W16 = l2W_ref[...].astype(BF)
    c3WT16 = c3WT_ref[...].astype(BF)
    l2b = l2b_ref[...]
    l1w = l1w_ref[...]
    l1b = l1b_ref[...]
    c3b = c3b_ref[...]
    mu = mu_ref[...]
    den = jnp.sqrt(var_ref[...] + 1e-5)
    gg = g_ref[...]
    be = be_ref[...]
    scols = []
    for k in range(KNN):
        g_k = g16_ref[k]
        ecol = jnp.exp(-dist_ref[0][:, k:k + 1])
        h = _hk(g_k, ecol, l2W16, l2b, l1w, l1b, c3WT16, c3b)
        hh = _leaky((h - mu) / den * gg + be)
        scols.append(jnp.max(hh, axis=1, keepdims=True))
    score = jnp.concatenate(scols, axis=1)
    idxs = idx_ref[0]
    it = jax.lax.broadcasted_iota(jnp.int32, (R, KNN), 1)
    x216 = x2_ref[0].astype(BF)
    iota = jax.lax.broadcasted_iota(jnp.int32, (R, N), 1)
    W116 = W1_ref[...].astype(BF)
    z1 = jnp.zeros((R, 512), F32)
    s = score
    for t in range(NKEY):
        m = jnp.max(s, axis=1, keepdims=True)
        a = jnp.min(jnp.where(s == m, it, KNN), axis=1, keepdims=True)
        ohk = it == a
        s = jnp.where(ohk, -BIG, s)
        tp = jnp.sum(jnp.where(ohk, idxs, 0), axis=1, keepdims=True)
        oh = (iota == tp).astype(BF)
        keyf = _dot(oh, x216).astype(BF)
        z1 = z1 + _dot(keyf, W116[t * 48:(t + 1) * 48, :])
    z1 = z1 + b1_ref[...]
    z1_ref[...] = z1
    bs = jnp.sum(z1, axis=0, keepdims=True)
    bsum_ref[0] = bs
    zc = z1 - bs / R
    bss_ref[0] = jnp.sum(zc * zc, axis=0, keepdims=True)


def _score_pass(g16, idx, dist, x2, l2W, l2b, l1W, l1b, c3WT, c3b, mu, var,
                c3g, c3be, m1W1, m1b1, R):
    B, N, _ = x2.shape
    NB = N // R
    grid = (B, NB)
    f = pl.pallas_call(
        functools.partial(_score_body, R, N),
        grid=grid,
        in_specs=[
            pl.BlockSpec((KNN, R, 48), lambda bb, nb, NB=NB: (0, bb * NB + nb, 0)),
            pl.BlockSpec((1, R, KNN), lambda bb, nb: (bb, nb, 0)),
            pl.BlockSpec((1, R, KNN), lambda bb, nb: (bb, nb, 0)),
            pl.BlockSpec((1, N, 48), lambda bb, nb: (bb, 0, 0)),
            pl.BlockSpec((48, 64), lambda bb, nb: (0, 0)),
            pl.BlockSpec((1, 64), lambda bb, nb: (0, 0)),
            pl.BlockSpec((1, 64), lambda bb, nb: (0, 0)),
            pl.BlockSpec((1, 64), lambda bb, nb: (0, 0)),
            pl.BlockSpec((128, 128), lambda bb, nb: (0, 0)),
            pl.BlockSpec((1, 128), lambda bb, nb: (0, 0)),
            pl.BlockSpec((1, 128), lambda bb, nb: (0, 0)),
            pl.BlockSpec((1, 128), lambda bb, nb: (0, 0)),
            pl.BlockSpec((1, 128), lambda bb, nb: (0, 0)),
            pl.BlockSpec((1, 128), lambda bb, nb: (0, 0)),
            pl.BlockSpec((480, 512), lambda bb, nb: (0, 0)),
            pl.BlockSpec((1, 512), lambda bb, nb: (0, 0)),
        ],
        out_specs=[
            pl.BlockSpec((R, 512), lambda bb, nb, NB=NB: (bb * NB + nb, 0)),
            pl.BlockSpec((1, 1, 512), lambda bb, nb, NB=NB: (bb * NB + nb, 0, 0)),
            pl.BlockSpec((1, 1, 512), lambda bb, nb, NB=NB: (bb * NB + nb, 0, 0)),
        ],
        out_shape=[
            jax.ShapeDtypeStruct((B * N, 512), F32),
            jax.ShapeDtypeStruct((B * NB, 1, 512), F32),
            jax.ShapeDtypeStruct((B * NB, 1, 512), F32),
        ],
    )
    return f(g16, idx, dist, x2, l2W, l2b.reshape(1, 64), l1W,
             l1b.reshape(1, 64), c3WT, c3b.reshape(1, 128), mu, var,
             c3g.reshape(1, 128), c3be.reshape(1, 128), m1W1,
             m1b1.reshape(1, 512))


def _mlp2_body(R, z1_ref, mu_ref, var_ref, g_ref, be_ref, W2_ref, b2_ref,
               z2_ref, bsum_ref, bss_ref):
    den = jnp.sqrt(var_ref[...] + 1e-5)
    a1 = _leaky((z1_ref[...] - mu_ref[...]) / den * g_ref[...] + be_ref[...])
    z2 = _dot(a1.astype(BF), W2_ref[...].astype(BF)) + b2_ref[...]
    z2_ref[...] = z2
    bs = jnp.sum(z2, axis=0, keepdims=True)
    bsum_ref[0] = bs
    zc = z2 - bs / R
    bss_ref[0] = jnp.sum(zc * zc, axis=0, keepdims=True)


def _mlp2_pass(z1, mu1, var1, g1, be1, m1W2, m1b2, R):
    M = z1.shape[0]
    NB = M // R
    f = pl.pallas_call(
        functools.partial(_mlp2_body, R),
        grid=(NB,),
        in_specs=[
            pl.BlockSpec((R, 512), lambda i: (i, 0)),
            pl.BlockSpec((1, 512), lambda i: (0, 0)),
            pl.BlockSpec((1, 512), lambda i: (0, 0)),
            pl.BlockSpec((1, 512), lambda i: (0, 0)),
            pl.BlockSpec((1, 512), lambda i: (0, 0)),
            pl.BlockSpec((512, 512), lambda i: (0, 0)),
            pl.BlockSpec((1, 512), lambda i: (0, 0)),
        ],
        out_specs=[
            pl.BlockSpec((R, 512), lambda i: (i, 0)),
            pl.BlockSpec((1, 1, 512), lambda i: (i, 0, 0)),
            pl.BlockSpec((1, 1, 512), lambda i: (i, 0, 0)),
        ],
        out_shape=[
            jax.ShapeDtypeStruct((M, 512), F32),
            jax.ShapeDtypeStruct((NB, 1, 512), F32),
            jax.ShapeDtypeStruct((NB, 1, 512), F32),
        ],
    )
    return f(z1, mu1, var1, g1.reshape(1, 512), be1.reshape(1, 512), m1W2,
             m1b2.reshape(1, 512))


def _final_body(z2_ref, mu_ref, var_ref, g_ref, be_ref, o_ref):
    den = jnp.sqrt(var_ref[...] + 1e-5)
    o_ref[...] = _leaky(
        (z2_ref[...] - mu_ref[...]) / den * g_ref[...] + be_ref[...])


def _final_pass(z2, mu2, var2, g2, be2, R):
    M = z2.shape[0]
    NB = M // R
    f = pl.pallas_call(
        _final_body,
        grid=(NB,),
        in_specs=[
            pl.BlockSpec((R, 512), lambda i: (i, 0)),
            pl.BlockSpec((1, 512), lambda i: (0, 0)),
            pl.BlockSpec((1, 512), lambda i: (0, 0)),
            pl.BlockSpec((1, 512), lambda i: (0, 0)),
            pl.BlockSpec((1, 512), lambda i: (0, 0)),
        ],
        out_specs=[pl.BlockSpec((R, 512), lambda i: (i, 0))],
        out_shape=[jax.ShapeDtypeStruct((M, 512), F32)],
    )
    return f(z2, mu2, var2, g2.reshape(1, 512), be2.reshape(1, 512))[0]


def _combine_stats(bsum, bss, m, cnt):
    bsum = bsum.reshape(bsum.shape[0], -1)
    bss = bss.reshape(bss.shape[0], -1)
    mu = jnp.sum(bsum, axis=0, keepdims=True) / cnt
    bm = bsum / m
    var = (jnp.sum(bss, axis=0, keepdims=True)
           + m * jnp.sum((bm - mu) ** 2, axis=0, keepdims=True)) / cnt
    return mu, var


def kernel(x, W1, b1, W2, b2, l1W, l1b, l2W, l2b, c3W, c3b, c3g, c3be,
           m1W1, m1b1, m1g1, m1be1, m1W2, m1b2, m1g2, m1be2):
    B, N, _ = x.shape
    R = 256
    Rk = 1024
    dist, idx, x1 = _knn_edge(x, W1, b1, Rk, emit_knn=True)
    x2, = _knn_edge(x1, W2, b2, Rk, emit_knn=False)

    c3WT = c3W.T
    g16, bsum, bss = _stats_pass(idx, dist, x2, l2W, l2b, l1W, l1b, c3WT,
                                 c3b, R)
    mu, var = _combine_stats(bsum, bss, KNN * R, B * N * KNN)

    z1, bsum1, bss1 = _score_pass(g16, idx, dist, x2, l2W, l2b, l1W, l1b,
                                  c3WT, c3b, mu, var, c3g, c3be, m1W1,
                                  m1b1, R)
    mu1, var1 = _combine_stats(bsum1, bss1, R, B * N)

    Rz = 512
    z2, bsum2, bss2 = _mlp2_pass(z1, mu1, var1, m1g1, m1be1, m1W2, m1b2, Rz)
    mu2, var2 = _combine_stats(bsum2, bss2, Rz, B * N)

    out = _final_pass(z2, mu2, var2, m1g2, m1be2, Rz)
    return out.reshape(B, N, 512)
